# baseline (speedup 1.0000x reference)
import functools

import jax
import jax.numpy as jnp
import jax.lax as lax
from jax.experimental import pallas as pl
from jax.experimental.pallas import tpu as pltpu

_B = 2
_N = 16384
_IN_C = 1
_NUM_CLASSES = 3
_NUM_POINTS = (4096, 1024, 512)
_SAMPLING = ("d-fps", "ctr_aware", "ctr_aware")
_NEIGHBORS = ((16, 32), (16, 32), (16, 32))
_RADII = ((0.2, 0.8), (0.8, 1.6), (1.6, 4.8))
_IN_LIST = (_IN_C, 64, 128)


def _gather_rows(x, idx):
    return jax.vmap(lambda xb, ib: xb[ib])(x, idx)


def _fps(xyz, npoint):
    b, n, _ = xyz.shape

    def body(i, state):
        cent, dist, far = state
        cent = cent.at[:, i].set(far)
        c = jnp.take_along_axis(xyz, far[:, None, None], axis=1)
        d = jnp.sum((xyz - c) ** 2, -1)
        dist = jnp.minimum(dist, d)
        far = jnp.argmax(dist, -1).astype(jnp.int32)
        return cent, dist, far

    cent = jnp.zeros((b, npoint), jnp.int32)
    dist = jnp.full((b, n), 1e10, jnp.float32)
    far = jnp.zeros((b,), jnp.int32)
    cent, _, _ = lax.fori_loop(0, npoint, body, (cent, dist, far))
    return cent


def _ball_query(dists, radius, nsample):
    n = dists.shape[-1]
    keyv = jnp.where(dists <= radius * radius,
                     jnp.arange(n, dtype=jnp.int32)[None, None, :], n)
    neg, _ = lax.top_k(-keyv, nsample)
    idx = -neg
    first = idx[:, :, :1]
    idx = jnp.where(idx == n, first, idx)
    idx = jnp.where(idx == n, 0, idx)
    return idx


def _identity_pallas(x):
    def body(x_ref, o_ref):
        o_ref[...] = x_ref[...]

    return pl.pallas_call(
        body,
        out_shape=jax.ShapeDtypeStruct(x.shape, x.dtype),
    )(x)


def _sa(points, feats_t, lp, li):
    npoint = _NUM_POINTS[li]
    cls_preds = None
    if _SAMPLING[li] == "ctr_aware":
        logits = feats_t @ lp["cls"]["W"].T + lp["cls"]["b"]
        scores = jnp.max(logits, -1)
        idx = lax.top_k(lax.stop_gradient(scores), npoint)[1]
        cls_preds = jnp.transpose(logits, (0, 2, 1))
    else:
        idx = _fps(lax.stop_gradient(points), npoint)
    new_xyz = _gather_rows(points, idx)
    dists = lax.stop_gradient(
        jnp.sum((new_xyz[:, :, None, :] - points[:, None, :, :]) ** 2, -1))
    outs = []
    for si, (r, ns) in enumerate(zip(_RADII[li], _NEIGHBORS[li])):
        nidx = _ball_query(dists, r, ns)
        g_xyz = _gather_rows(points, nidx) - new_xyz[:, :, None, :]
        g_feat = _gather_rows(feats_t, nidx)
        h = jnp.concatenate([g_xyz, g_feat], -1)
        for conv in lp["mlps"][si]:
            h = jax.nn.relu(h @ conv["W"].T + conv["b"])
        outs.append(jnp.max(h, axis=2))
    cat = jnp.concatenate(outs, -1)
    new_feat = jax.nn.relu(cat @ lp["agg"]["W"].T + lp["agg"]["b"])
    return new_xyz, new_feat, cls_preds


def kernel(points, features, params):
    feats_t = jnp.transpose(features, (0, 2, 1))
    feats_t = _identity_pallas(feats_t)
    cls_list = []
    pts_list = []
    for li in range(3):
        ip = points
        points, feats_t, cp = _sa(points, feats_t, params["layers"][li], li)
        if cp is not None:
            cls_list.append(cp)
            pts_list.append(ip)
    return points, jnp.transpose(feats_t, (0, 2, 1)), cls_list, pts_list


# speedup vs baseline: 1.4377x; 1.4377x over previous
import functools

import jax
import jax.numpy as jnp
import jax.lax as lax
from jax.experimental import pallas as pl
from jax.experimental.pallas import tpu as pltpu

_B = 2
_N = 16384
_IN_C = 1
_NUM_CLASSES = 3
_NUM_POINTS = (4096, 1024, 512)
_SAMPLING = ("d-fps", "ctr_aware", "ctr_aware")
_NEIGHBORS = ((16, 32), (16, 32), (16, 32))
_RADII = ((0.2, 0.8), (0.8, 1.6), (1.6, 4.8))
_IN_LIST = (_IN_C, 64, 128)


def _gather_rows(x, idx):
    return jax.vmap(lambda xb, ib: xb[ib])(x, idx)


def _fps_body(npoint, n_rows, x_ref, o_ref):
    rows = n_rows
    iota_r = lax.broadcasted_iota(jnp.int32, (rows, 128), 0)
    iota_c = lax.broadcasted_iota(jnp.int32, (rows, 128), 1)
    gidx = iota_r * 128 + iota_c
    lane = lax.broadcasted_iota(jnp.int32, (1, 128), 1)

    x = x_ref[0, 0, :, :]
    y = x_ref[0, 1, :, :]
    z = x_ref[0, 2, :, :]

    def body(i, carry):
        dist, far = carry
        r = far // 128
        c = far % 128
        lmask = lane == c
        cx = jnp.sum(jnp.where(lmask, x_ref[0, 0, pl.ds(r, 1), :], 0.0))
        cy = jnp.sum(jnp.where(lmask, x_ref[0, 1, pl.ds(r, 1), :], 0.0))
        cz = jnp.sum(jnp.where(lmask, x_ref[0, 2, pl.ds(r, 1), :], 0.0))
        crow = jnp.where(lane == 0, cx,
                         jnp.where(lane == 1, cy,
                                   jnp.where(lane == 2, cz, 0.0)))
        o_ref[0, pl.ds(i, 1), :] = crow
        dx = x - cx
        dy = y - cy
        dz = z - cz
        d = dx * dx + dy * dy + dz * dz
        dist = jnp.minimum(dist, d)
        m = jnp.max(dist)
        far2 = jnp.min(jnp.where(dist == m, gidx, jnp.int32(1 << 30)))
        return dist, far2

    dist0 = jnp.full((rows, 128), 1e10, jnp.float32)
    lax.fori_loop(0, npoint, body, (dist0, jnp.int32(0)))


def _fps_pallas(points, npoint):
    b, n, _ = points.shape
    rows = n // 128
    pts = jnp.transpose(points, (0, 2, 1)).reshape(b, 3, rows, 128)
    out = pl.pallas_call(
        functools.partial(_fps_body, npoint, rows),
        grid=(b,),
        in_specs=[pl.BlockSpec((1, 3, rows, 128), lambda i: (i, 0, 0, 0))],
        out_specs=pl.BlockSpec((1, npoint, 128), lambda i: (i, 0, 0)),
        out_shape=jax.ShapeDtypeStruct((b, npoint, 128), jnp.float32),
    )(pts)
    return out[:, :, :3]


def _fps(xyz, npoint):
    b, n, _ = xyz.shape

    def body(i, state):
        cent, dist, far = state
        cent = cent.at[:, i].set(far)
        c = jnp.take_along_axis(xyz, far[:, None, None], axis=1)
        d = jnp.sum((xyz - c) ** 2, -1)
        dist = jnp.minimum(dist, d)
        far = jnp.argmax(dist, -1).astype(jnp.int32)
        return cent, dist, far

    cent = jnp.zeros((b, npoint), jnp.int32)
    dist = jnp.full((b, n), 1e10, jnp.float32)
    far = jnp.zeros((b,), jnp.int32)
    cent, _, _ = lax.fori_loop(0, npoint, body, (cent, dist, far))
    return cent


def _ball_query(dists, radius, nsample):
    n = dists.shape[-1]
    keyv = jnp.where(dists <= radius * radius,
                     jnp.arange(n, dtype=jnp.int32)[None, None, :], n)
    neg, _ = lax.top_k(-keyv, nsample)
    idx = -neg
    first = idx[:, :, :1]
    idx = jnp.where(idx == n, first, idx)
    idx = jnp.where(idx == n, 0, idx)
    return idx


def _identity_pallas(x):
    def body(x_ref, o_ref):
        o_ref[...] = x_ref[...]

    return pl.pallas_call(
        body,
        out_shape=jax.ShapeDtypeStruct(x.shape, x.dtype),
    )(x)


def _sa(points, feats_t, lp, li):
    npoint = _NUM_POINTS[li]
    cls_preds = None
    if _SAMPLING[li] == "ctr_aware":
        logits = feats_t @ lp["cls"]["W"].T + lp["cls"]["b"]
        scores = jnp.max(logits, -1)
        idx = lax.top_k(lax.stop_gradient(scores), npoint)[1]
        cls_preds = jnp.transpose(logits, (0, 2, 1))
        new_xyz = _gather_rows(points, idx)
    else:
        new_xyz = _fps_pallas(lax.stop_gradient(points), npoint)
    dists = lax.stop_gradient(
        jnp.sum((new_xyz[:, :, None, :] - points[:, None, :, :]) ** 2, -1))
    outs = []
    for si, (r, ns) in enumerate(zip(_RADII[li], _NEIGHBORS[li])):
        nidx = _ball_query(dists, r, ns)
        g_xyz = _gather_rows(points, nidx) - new_xyz[:, :, None, :]
        g_feat = _gather_rows(feats_t, nidx)
        h = jnp.concatenate([g_xyz, g_feat], -1)
        for conv in lp["mlps"][si]:
            h = jax.nn.relu(h @ conv["W"].T + conv["b"])
        outs.append(jnp.max(h, axis=2))
    cat = jnp.concatenate(outs, -1)
    new_feat = jax.nn.relu(cat @ lp["agg"]["W"].T + lp["agg"]["b"])
    return new_xyz, new_feat, cls_preds


def kernel(points, features, params):
    feats_t = jnp.transpose(features, (0, 2, 1))
    feats_t = _identity_pallas(feats_t)
    cls_list = []
    pts_list = []
    for li in range(3):
        ip = points
        points, feats_t, cp = _sa(points, feats_t, params["layers"][li], li)
        if cp is not None:
            cls_list.append(cp)
            pts_list.append(ip)
    return points, jnp.transpose(feats_t, (0, 2, 1)), cls_list, pts_list


# speedup vs baseline: 6.3648x; 4.4269x over previous
import functools

import jax
import jax.numpy as jnp
import jax.lax as lax
import numpy as np
from jax.experimental import pallas as pl
from jax.experimental.pallas import tpu as pltpu
from jax.experimental.pallas import tpu_sc as plsc

_B = 2
_N = 16384
_IN_C = 1
_NUM_CLASSES = 3
_NUM_POINTS = (4096, 1024, 512)
_SAMPLING = ("d-fps", "ctr_aware", "ctr_aware")
_NEIGHBORS = ((16, 32), (16, 32), (16, 32))
_RADII = ((0.2, 0.8), (0.8, 1.6), (1.6, 4.8))
_IN_LIST = (_IN_C, 64, 128)


def _gather_rows(x, idx):
    return jax.vmap(lambda xb, ib: xb[ib])(x, idx)


def _fps_body(npoint, n_rows, x_ref, o_ref):
    rows = n_rows
    iota_r = lax.broadcasted_iota(jnp.int32, (rows, 128), 0)
    iota_c = lax.broadcasted_iota(jnp.int32, (rows, 128), 1)
    gidx = iota_r * 128 + iota_c
    lane = lax.broadcasted_iota(jnp.int32, (1, 128), 1)

    x = x_ref[0, 0, :, :]
    y = x_ref[0, 1, :, :]
    z = x_ref[0, 2, :, :]

    def body(i, carry):
        dist, far = carry
        r = far // 128
        c = far % 128
        lmask = lane == c
        cx = jnp.sum(jnp.where(lmask, x_ref[0, 0, pl.ds(r, 1), :], 0.0))
        cy = jnp.sum(jnp.where(lmask, x_ref[0, 1, pl.ds(r, 1), :], 0.0))
        cz = jnp.sum(jnp.where(lmask, x_ref[0, 2, pl.ds(r, 1), :], 0.0))
        crow = jnp.where(lane == 0, cx,
                         jnp.where(lane == 1, cy,
                                   jnp.where(lane == 2, cz, 0.0)))
        o_ref[0, pl.ds(i, 1), :] = crow
        dx = x - cx
        dy = y - cy
        dz = z - cz
        d = dx * dx + dy * dy + dz * dz
        dist = jnp.minimum(dist, d)
        m = jnp.max(dist)
        far2 = jnp.min(jnp.where(dist == m, gidx, jnp.int32(1 << 30)))
        return dist, far2

    dist0 = jnp.full((rows, 128), 1e10, jnp.float32)
    lax.fori_loop(0, npoint, body, (dist0, jnp.int32(0)))


def _fps_pallas(points, npoint):
    b, n, _ = points.shape
    rows = n // 128
    pts = jnp.transpose(points, (0, 2, 1)).reshape(b, 3, rows, 128)
    out = pl.pallas_call(
        functools.partial(_fps_body, npoint, rows),
        grid=(b,),
        in_specs=[pl.BlockSpec((1, 3, rows, 128), lambda i: (i, 0, 0, 0))],
        out_specs=pl.BlockSpec((1, npoint, 128), lambda i: (i, 0, 0)),
        out_shape=jax.ShapeDtypeStruct((b, npoint, 128), jnp.float32),
    )(pts)
    return out[:, :, :3]


def _fps(xyz, npoint):
    b, n, _ = xyz.shape

    def body(i, state):
        cent, dist, far = state
        cent = cent.at[:, i].set(far)
        c = jnp.take_along_axis(xyz, far[:, None, None], axis=1)
        d = jnp.sum((xyz - c) ** 2, -1)
        dist = jnp.minimum(dist, d)
        far = jnp.argmax(dist, -1).astype(jnp.int32)
        return cent, dist, far

    cent = jnp.zeros((b, npoint), jnp.int32)
    dist = jnp.full((b, n), 1e10, jnp.float32)
    far = jnp.zeros((b,), jnp.int32)
    cent, _, _ = lax.fori_loop(0, npoint, body, (cent, dist, far))
    return cent


def _bq_mask_body(nc, r0sq, r1sq, nchunks, q_ref, p_ref,
                  w0_ref, w1_ref, c0_ref, c1_ref, cs_ref):
    wc = nc // 16
    qx = q_ref[0, :, 0:1]
    qy = q_ref[0, :, 1:2]
    qz = q_ref[0, :, 2:3]
    jidx = lax.broadcasted_iota(jnp.int32, (nc, wc), 0)
    widx = lax.broadcasted_iota(jnp.int32, (nc, wc), 1)
    blk = jidx // 16 == widx
    packm = jnp.where(blk, (jnp.int32(1) << (jidx % 16)), 0).astype(jnp.bfloat16)
    onesm = jnp.where(blk, 1, 0).astype(jnp.bfloat16)
    ta = lax.broadcasted_iota(jnp.int32, (wc, wc), 0)
    tb = lax.broadcasted_iota(jnp.int32, (wc, wc), 1)
    tri = jnp.where(ta <= tb, 1, 0).astype(jnp.bfloat16)

    cs_ref[:, :] = jnp.zeros((128, 2), jnp.float32)

    def chunk(k, _):
        base = pl.multiple_of(k * nc, nc)
        px = p_ref[0, 0:1, pl.ds(base, nc)]
        py = p_ref[0, 1:2, pl.ds(base, nc)]
        pz = p_ref[0, 2:3, pl.ds(base, nc)]
        dx = qx - px
        dy = qy - py
        dz = qz - pz
        d = dx * dx + dy * dy + dz * dz
        v0 = (d <= r0sq).astype(jnp.bfloat16)
        v1 = (d <= r1sq).astype(jnp.bfloat16)
        w0 = jax.lax.dot(v0, packm, preferred_element_type=jnp.float32)
        w1 = jax.lax.dot(v1, packm, preferred_element_type=jnp.float32)
        pc0 = jax.lax.dot(v0, onesm, preferred_element_type=jnp.float32)
        pc1 = jax.lax.dot(v1, onesm, preferred_element_type=jnp.float32)
        c0 = jax.lax.dot(pc0.astype(jnp.bfloat16), tri,
                         preferred_element_type=jnp.float32) + cs_ref[:, 0:1]
        c1 = jax.lax.dot(pc1.astype(jnp.bfloat16), tri,
                         preferred_element_type=jnp.float32) + cs_ref[:, 1:2]
        obase = pl.multiple_of(k * wc, wc)
        w0_ref[0, :, pl.ds(obase, wc)] = w0.astype(jnp.int32)
        w1_ref[0, :, pl.ds(obase, wc)] = w1.astype(jnp.int32)
        c0_ref[0, :, pl.ds(obase, wc)] = c0.astype(jnp.int32)
        c1_ref[0, :, pl.ds(obase, wc)] = c1.astype(jnp.int32)
        cs_ref[:, 0:1] = c0[:, wc - 1:wc]
        cs_ref[:, 1:2] = c1[:, wc - 1:wc]
        return 0

    lax.fori_loop(0, nchunks, chunk, 0)


def _bq_masks(new_xyz, pts_t, r0, r1):
    b, m, _ = new_xyz.shape
    n = pts_t.shape[-1]
    nc = min(n, 2048)
    nchunks = n // nc
    w = n // 16
    r0sq = np.float32(r0 * r0)
    r1sq = np.float32(r1 * r1)
    shp = jax.ShapeDtypeStruct((b, m, w), jnp.int32)
    out = pl.pallas_call(
        functools.partial(_bq_mask_body, nc, r0sq, r1sq, nchunks),
        grid=(b, m // 128),
        in_specs=[
            pl.BlockSpec((1, 128, 3), lambda i, j: (i, j, 0)),
            pl.BlockSpec((1, 3, n), lambda i, j: (i, 0, 0)),
        ],
        out_specs=[
            pl.BlockSpec((1, 128, w), lambda i, j: (i, j, 0)),
            pl.BlockSpec((1, 128, w), lambda i, j: (i, j, 0)),
            pl.BlockSpec((1, 128, w), lambda i, j: (i, j, 0)),
            pl.BlockSpec((1, 128, w), lambda i, j: (i, j, 0)),
        ],
        out_shape=[shp, shp, shp, shp],
        scratch_shapes=[pltpu.VMEM((128, 2), jnp.float32)],
    )(new_xyz, pts_t)
    flat = [x.reshape(b * m * w) for x in out]
    return (flat[0], flat[2]), (flat[1], flat[3])


def _bq_extract_sc(wc_pair, ns, w, m_per_batch, n_pts):
    words_flat, c_flat = wc_pair
    rows = words_flat.shape[0] // w
    mesh = plsc.VectorSubcoreMesh(core_axis_name="c", subcore_axis_name="s")
    info = plsc.get_sparse_core_info()
    nw = info.num_cores * info.num_subcores
    qpw = rows // nw
    lanes = 512
    nch = lanes // 128
    qb = lanes // ns
    nbatches = qpw // qb
    log2ns = ns.bit_length() - 1
    log2m = m_per_batch.bit_length() - 1
    strides = []
    st = w // 2
    while st >= 1:
        strides.append(st)
        st //= 2

    @functools.partial(
        pl.kernel, mesh=mesh,
        out_type=jax.ShapeDtypeStruct((rows * ns,), jnp.int32),
        scratch_types=[
            pltpu.VMEM((nch, 128), jnp.int32),
            pltpu.VMEM((nch, 128), jnp.int32),
            pltpu.VMEM((lanes,), jnp.int32),
            pltpu.VMEM((lanes,), jnp.int32),
            pltpu.VMEM((lanes,), jnp.int32),
            pltpu.VMEM((lanes,), jnp.int32),
            pltpu.VMEM((lanes,), jnp.int32),
            pltpu.SemaphoreType.DMA,
        ],
    )
    def k(wf, cf, out, idxb, gbuf, posb, cbb, seb, tbb, obuf, sem):
        wid = lax.axis_index("s") * info.num_cores + lax.axis_index("c")
        base_q = wid * qpw
        lane = lax.iota(jnp.int32, 16)
        nv = lanes // 16

        def gather_round(src):
            cps = [pltpu.async_copy(src.at[idxb.at[c]], gbuf.at[c], sem)
                   for c in range(nch)]
            for cp in cps:
                cp.wait()

        def batch(bi, _):
            q0 = base_q + bi * qb

            def init_idx(v, _):
                fp = v * 16 + lane
                iq = fp >> log2ns
                idxb[v // 8, pl.ds((v % 8) * 16, 16)] = (q0 + iq) * w + (w - 1)
                return 0

            lax.fori_loop(0, nv, init_idx, 0)
            gather_round(cf)

            def init2(v, _):
                fp = v * 16 + lane
                s = fp & (ns - 1)
                tt = gbuf[v // 8, pl.ds((v % 8) * 16, 16)]
                se = jnp.maximum(0, jnp.minimum(s, tt - 1))
                seb[pl.ds(v * 16, 16)] = se
                tbb[pl.ds(v * 16, 16)] = tt
                posb[pl.ds(v * 16, 16)] = jnp.zeros((16,), jnp.int32)
                cbb[pl.ds(v * 16, 16)] = jnp.zeros((16,), jnp.int32)
                return 0

            lax.fori_loop(0, nv, init2, 0)

            for stv in strides:
                def mkidx(v, _):
                    fp = v * 16 + lane
                    iq = fp >> log2ns
                    npos = posb[pl.ds(v * 16, 16)] + stv
                    idxb[v // 8, pl.ds((v % 8) * 16, 16)] = \
                        (q0 + iq) * w + npos - 1
                    return 0

                lax.fori_loop(0, nv, mkidx, 0)
                gather_round(cf)

                def upd(v, _):
                    cv = gbuf[v // 8, pl.ds((v % 8) * 16, 16)]
                    se = seb[pl.ds(v * 16, 16)]
                    pos = posb[pl.ds(v * 16, 16)]
                    ok = cv <= se
                    posb[pl.ds(v * 16, 16)] = jnp.where(ok, pos + stv, pos)
                    cb = cbb[pl.ds(v * 16, 16)]
                    cbb[pl.ds(v * 16, 16)] = jnp.where(ok, cv, cb)
                    return 0

                lax.fori_loop(0, nv, upd, 0)

            def widx(v, _):
                fp = v * 16 + lane
                iq = fp >> log2ns
                pos = posb[pl.ds(v * 16, 16)]
                idxb[v // 8, pl.ds((v % 8) * 16, 16)] = (q0 + iq) * w + pos
                return 0

            lax.fori_loop(0, nv, widx, 0)
            gather_round(wf)

            def fin(v, _):
                word = gbuf[v // 8, pl.ds((v % 8) * 16, 16)]
                se = seb[pl.ds(v * 16, 16)]
                cb = cbb[pl.ds(v * 16, 16)]
                pos = posb[pl.ds(v * 16, 16)]
                tt = tbb[pl.ds(v * 16, 16)]
                kloc = se - cb
                for t in range(15):
                    word = jnp.where(t < kloc, word & (word - 1), word)
                bb = word & (-word)
                ee = (jnp.where((bb & 0xAAAA) != 0, 1, 0)
                      + jnp.where((bb & 0xCCCC) != 0, 2, 0)
                      + jnp.where((bb & 0xF0F0) != 0, 4, 0)
                      + jnp.where((bb & 0xFF00) != 0, 8, 0))
                fp = v * 16 + lane
                iq = fp >> log2ns
                row = q0 + iq
                fa = (row >> log2m) * n_pts
                val = pos * 16 + ee + fa
                obuf[pl.ds(v * 16, 16)] = jnp.where(tt > 0, val, fa)
                return 0

            lax.fori_loop(0, nv, fin, 0)
            pltpu.sync_copy(obuf, out.at[pl.ds(q0 * ns, lanes)])
            return 0

        lax.fori_loop(0, nbatches, batch, 0)

    return k(words_flat, c_flat)


def _ball_query(dists, radius, nsample):
    n = dists.shape[-1]
    keyv = jnp.where(dists <= radius * radius,
                     jnp.arange(n, dtype=jnp.int32)[None, None, :], n)
    neg, _ = lax.top_k(-keyv, nsample)
    idx = -neg
    first = idx[:, :, :1]
    idx = jnp.where(idx == n, first, idx)
    idx = jnp.where(idx == n, 0, idx)
    return idx


def _identity_pallas(x):
    def body(x_ref, o_ref):
        o_ref[...] = x_ref[...]

    return pl.pallas_call(
        body,
        out_shape=jax.ShapeDtypeStruct(x.shape, x.dtype),
    )(x)


def _sa(points, feats_t, lp, li):
    npoint = _NUM_POINTS[li]
    cls_preds = None
    if _SAMPLING[li] == "ctr_aware":
        logits = feats_t @ lp["cls"]["W"].T + lp["cls"]["b"]
        scores = jnp.max(logits, -1)
        idx = lax.top_k(lax.stop_gradient(scores), npoint)[1]
        cls_preds = jnp.transpose(logits, (0, 2, 1))
        new_xyz = _gather_rows(points, idx)
    else:
        new_xyz = _fps_pallas(lax.stop_gradient(points), npoint)
    b, n, _ = points.shape
    pts_t3 = jnp.transpose(points, (0, 2, 1))
    words = _bq_masks(new_xyz, pts_t3, _RADII[li][0], _RADII[li][1])
    boff = (jnp.arange(b, dtype=jnp.int32) * n)[:, None, None]
    outs = []
    for si, (r, ns) in enumerate(zip(_RADII[li], _NEIGHBORS[li])):
        flat = _bq_extract_sc(words[si], ns, n // 16, npoint, n)
        nidx = flat.reshape(b, npoint, ns) - boff
        g_xyz = _gather_rows(points, nidx) - new_xyz[:, :, None, :]
        g_feat = _gather_rows(feats_t, nidx)
        h = jnp.concatenate([g_xyz, g_feat], -1)
        for conv in lp["mlps"][si]:
            h = jax.nn.relu(h @ conv["W"].T + conv["b"])
        outs.append(jnp.max(h, axis=2))
    cat = jnp.concatenate(outs, -1)
    new_feat = jax.nn.relu(cat @ lp["agg"]["W"].T + lp["agg"]["b"])
    return new_xyz, new_feat, cls_preds


def kernel(points, features, params):
    feats_t = jnp.transpose(features, (0, 2, 1))
    feats_t = _identity_pallas(feats_t)
    cls_list = []
    pts_list = []
    for li in range(3):
        ip = points
        points, feats_t, cp = _sa(points, feats_t, params["layers"][li], li)
        if cp is not None:
            cls_list.append(cp)
            pts_list.append(ip)
    return points, jnp.transpose(feats_t, (0, 2, 1)), cls_list, pts_list


# speedup vs baseline: 22.6333x; 3.5560x over previous
import functools

import jax
import jax.numpy as jnp
import jax.lax as lax
import numpy as np
from jax.experimental import pallas as pl
from jax.experimental.pallas import tpu as pltpu
from jax.experimental.pallas import tpu_sc as plsc

_B = 2
_N = 16384
_IN_C = 1
_NUM_CLASSES = 3
_NUM_POINTS = (4096, 1024, 512)
_SAMPLING = ("d-fps", "ctr_aware", "ctr_aware")
_NEIGHBORS = ((16, 32), (16, 32), (16, 32))
_RADII = ((0.2, 0.8), (0.8, 1.6), (1.6, 4.8))
_IN_LIST = (_IN_C, 64, 128)


def _gather_rows(x, idx):
    return jax.vmap(lambda xb, ib: xb[ib])(x, idx)


def _fps_body(npoint, n_rows, x_ref, o_ref):
    rows = n_rows
    iota_r = lax.broadcasted_iota(jnp.int32, (rows, 128), 0)
    iota_c = lax.broadcasted_iota(jnp.int32, (rows, 128), 1)
    gidx = iota_r * 128 + iota_c
    lane = lax.broadcasted_iota(jnp.int32, (1, 128), 1)

    x = x_ref[0, 0, :, :]
    y = x_ref[0, 1, :, :]
    z = x_ref[0, 2, :, :]

    def body(i, carry):
        dist, far = carry
        r = far // 128
        c = far % 128
        lmask = lane == c
        cx = jnp.sum(jnp.where(lmask, x_ref[0, 0, pl.ds(r, 1), :], 0.0))
        cy = jnp.sum(jnp.where(lmask, x_ref[0, 1, pl.ds(r, 1), :], 0.0))
        cz = jnp.sum(jnp.where(lmask, x_ref[0, 2, pl.ds(r, 1), :], 0.0))
        crow = jnp.where(lane == 0, cx,
                         jnp.where(lane == 1, cy,
                                   jnp.where(lane == 2, cz, 0.0)))
        o_ref[0, pl.ds(i, 1), :] = crow
        dx = x - cx
        dy = y - cy
        dz = z - cz
        d = dx * dx + dy * dy + dz * dz
        dist = jnp.minimum(dist, d)
        m = jnp.max(dist)
        far2 = jnp.min(jnp.where(dist == m, gidx, jnp.int32(1 << 30)))
        return dist, far2

    dist0 = jnp.full((rows, 128), 1e10, jnp.float32)
    lax.fori_loop(0, npoint, body, (dist0, jnp.int32(0)))


def _fps_pallas(points, npoint):
    b, n, _ = points.shape
    rows = n // 128
    pts = jnp.transpose(points, (0, 2, 1)).reshape(b, 3, rows, 128)
    out = pl.pallas_call(
        functools.partial(_fps_body, npoint, rows),
        grid=(b,),
        in_specs=[pl.BlockSpec((1, 3, rows, 128), lambda i: (i, 0, 0, 0))],
        out_specs=pl.BlockSpec((1, npoint, 128), lambda i: (i, 0, 0)),
        out_shape=jax.ShapeDtypeStruct((b, npoint, 128), jnp.float32),
    )(pts)
    return out[:, :, :3]


def _fps(xyz, npoint):
    b, n, _ = xyz.shape

    def body(i, state):
        cent, dist, far = state
        cent = cent.at[:, i].set(far)
        c = jnp.take_along_axis(xyz, far[:, None, None], axis=1)
        d = jnp.sum((xyz - c) ** 2, -1)
        dist = jnp.minimum(dist, d)
        far = jnp.argmax(dist, -1).astype(jnp.int32)
        return cent, dist, far

    cent = jnp.zeros((b, npoint), jnp.int32)
    dist = jnp.full((b, n), 1e10, jnp.float32)
    far = jnp.zeros((b,), jnp.int32)
    cent, _, _ = lax.fori_loop(0, npoint, body, (cent, dist, far))
    return cent


def _bq_mask_body(nc, r0sq, r1sq, nchunks, q_ref, p_ref,
                  w0_ref, w1_ref, c0_ref, c1_ref, cs_ref):
    wc = nc // 16
    qx = q_ref[0, :, 0:1]
    qy = q_ref[0, :, 1:2]
    qz = q_ref[0, :, 2:3]
    jidx = lax.broadcasted_iota(jnp.int32, (nc, wc), 0)
    widx = lax.broadcasted_iota(jnp.int32, (nc, wc), 1)
    blk = jidx // 16 == widx
    packm = jnp.where(blk, (jnp.int32(1) << (jidx % 16)), 0).astype(jnp.bfloat16)
    onesm = jnp.where(blk, 1, 0).astype(jnp.bfloat16)
    ta = lax.broadcasted_iota(jnp.int32, (wc, wc), 0)
    tb = lax.broadcasted_iota(jnp.int32, (wc, wc), 1)
    tri = jnp.where(ta <= tb, 1, 0).astype(jnp.bfloat16)

    cs_ref[:, :] = jnp.zeros((128, 2), jnp.float32)

    def chunk(k, _):
        base = pl.multiple_of(k * nc, nc)
        px = p_ref[0, 0:1, pl.ds(base, nc)]
        py = p_ref[0, 1:2, pl.ds(base, nc)]
        pz = p_ref[0, 2:3, pl.ds(base, nc)]
        dx = qx - px
        dy = qy - py
        dz = qz - pz
        d = dx * dx + dy * dy + dz * dz
        v0 = (d <= r0sq).astype(jnp.bfloat16)
        v1 = (d <= r1sq).astype(jnp.bfloat16)
        w0 = jax.lax.dot(v0, packm, preferred_element_type=jnp.float32)
        w1 = jax.lax.dot(v1, packm, preferred_element_type=jnp.float32)
        pc0 = jax.lax.dot(v0, onesm, preferred_element_type=jnp.float32)
        pc1 = jax.lax.dot(v1, onesm, preferred_element_type=jnp.float32)
        c0 = jax.lax.dot(pc0.astype(jnp.bfloat16), tri,
                         preferred_element_type=jnp.float32) + cs_ref[:, 0:1]
        c1 = jax.lax.dot(pc1.astype(jnp.bfloat16), tri,
                         preferred_element_type=jnp.float32) + cs_ref[:, 1:2]
        obase = pl.multiple_of(k * wc, wc)
        w0_ref[0, :, pl.ds(obase, wc)] = w0.astype(jnp.int32)
        w1_ref[0, :, pl.ds(obase, wc)] = w1.astype(jnp.int32)
        c0_ref[0, :, pl.ds(obase, wc)] = c0.astype(jnp.int32)
        c1_ref[0, :, pl.ds(obase, wc)] = c1.astype(jnp.int32)
        cs_ref[:, 0:1] = c0[:, wc - 1:wc]
        cs_ref[:, 1:2] = c1[:, wc - 1:wc]
        return 0

    lax.fori_loop(0, nchunks, chunk, 0)


def _bq_masks(new_xyz, pts_t, r0, r1):
    b, m, _ = new_xyz.shape
    n = pts_t.shape[-1]
    nc = min(n, 2048)
    nchunks = n // nc
    w = n // 16
    r0sq = np.float32(r0 * r0)
    r1sq = np.float32(r1 * r1)
    shp = jax.ShapeDtypeStruct((b, m, w), jnp.int32)
    out = pl.pallas_call(
        functools.partial(_bq_mask_body, nc, r0sq, r1sq, nchunks),
        grid=(b, m // 128),
        in_specs=[
            pl.BlockSpec((1, 128, 3), lambda i, j: (i, j, 0)),
            pl.BlockSpec((1, 3, n), lambda i, j: (i, 0, 0)),
        ],
        out_specs=[
            pl.BlockSpec((1, 128, w), lambda i, j: (i, j, 0)),
            pl.BlockSpec((1, 128, w), lambda i, j: (i, j, 0)),
            pl.BlockSpec((1, 128, w), lambda i, j: (i, j, 0)),
            pl.BlockSpec((1, 128, w), lambda i, j: (i, j, 0)),
        ],
        out_shape=[shp, shp, shp, shp],
        scratch_shapes=[pltpu.VMEM((128, 2), jnp.float32)],
    )(new_xyz, pts_t)
    flat = [x.reshape(b * m * w) for x in out]
    return (flat[0], flat[2]), (flat[1], flat[3])


def _bq_extract_sc(wc_pair, ns, w, m_per_batch, n_pts):
    words_flat, c_flat = wc_pair
    rows = words_flat.shape[0] // w
    mesh = plsc.VectorSubcoreMesh(core_axis_name="c", subcore_axis_name="s")
    info = plsc.get_sparse_core_info()
    nw = info.num_cores * info.num_subcores
    qpw = rows // nw
    lanes = 512
    nch = lanes // 128
    qb = lanes // ns
    nbatches = qpw // qb
    log2ns = ns.bit_length() - 1
    log2m = m_per_batch.bit_length() - 1
    strides = []
    st = w // 2
    while st >= 1:
        strides.append(st)
        st //= 2

    @functools.partial(
        pl.kernel, mesh=mesh,
        out_type=jax.ShapeDtypeStruct((rows * ns,), jnp.int32),
        scratch_types=[
            pltpu.VMEM((nch, 128), jnp.int32),
            pltpu.VMEM((nch, 128), jnp.int32),
            pltpu.VMEM((lanes,), jnp.int32),
            pltpu.VMEM((lanes,), jnp.int32),
            pltpu.VMEM((lanes,), jnp.int32),
            pltpu.VMEM((lanes,), jnp.int32),
            pltpu.VMEM((lanes,), jnp.int32),
            pltpu.SemaphoreType.DMA,
        ],
    )
    def k(wf, cf, out, idxb, gbuf, posb, cbb, seb, tbb, obuf, sem):
        wid = lax.axis_index("s") * info.num_cores + lax.axis_index("c")
        base_q = wid * qpw
        lane = lax.iota(jnp.int32, 16)
        nv = lanes // 16

        def gather_round(src):
            cps = [pltpu.async_copy(src.at[idxb.at[c]], gbuf.at[c], sem)
                   for c in range(nch)]
            for cp in cps:
                cp.wait()

        def batch(bi, _):
            q0 = base_q + bi * qb

            def init_idx(v, _):
                fp = v * 16 + lane
                iq = fp >> log2ns
                idxb[v // 8, pl.ds((v % 8) * 16, 16)] = (q0 + iq) * w + (w - 1)
                return 0

            lax.fori_loop(0, nv, init_idx, 0)
            gather_round(cf)

            def init2(v, _):
                fp = v * 16 + lane
                s = fp & (ns - 1)
                tt = gbuf[v // 8, pl.ds((v % 8) * 16, 16)]
                se = jnp.maximum(0, jnp.minimum(s, tt - 1))
                seb[pl.ds(v * 16, 16)] = se
                tbb[pl.ds(v * 16, 16)] = tt
                posb[pl.ds(v * 16, 16)] = jnp.zeros((16,), jnp.int32)
                cbb[pl.ds(v * 16, 16)] = jnp.zeros((16,), jnp.int32)
                return 0

            lax.fori_loop(0, nv, init2, 0)

            for stv in strides:
                def mkidx(v, _):
                    fp = v * 16 + lane
                    iq = fp >> log2ns
                    npos = posb[pl.ds(v * 16, 16)] + stv
                    idxb[v // 8, pl.ds((v % 8) * 16, 16)] = \
                        (q0 + iq) * w + npos - 1
                    return 0

                lax.fori_loop(0, nv, mkidx, 0)
                gather_round(cf)

                def upd(v, _):
                    cv = gbuf[v // 8, pl.ds((v % 8) * 16, 16)]
                    se = seb[pl.ds(v * 16, 16)]
                    pos = posb[pl.ds(v * 16, 16)]
                    ok = cv <= se
                    posb[pl.ds(v * 16, 16)] = jnp.where(ok, pos + stv, pos)
                    cb = cbb[pl.ds(v * 16, 16)]
                    cbb[pl.ds(v * 16, 16)] = jnp.where(ok, cv, cb)
                    return 0

                lax.fori_loop(0, nv, upd, 0)

            def widx(v, _):
                fp = v * 16 + lane
                iq = fp >> log2ns
                pos = posb[pl.ds(v * 16, 16)]
                idxb[v // 8, pl.ds((v % 8) * 16, 16)] = (q0 + iq) * w + pos
                return 0

            lax.fori_loop(0, nv, widx, 0)
            gather_round(wf)

            def fin(v, _):
                word = gbuf[v // 8, pl.ds((v % 8) * 16, 16)]
                se = seb[pl.ds(v * 16, 16)]
                cb = cbb[pl.ds(v * 16, 16)]
                pos = posb[pl.ds(v * 16, 16)]
                tt = tbb[pl.ds(v * 16, 16)]
                kloc = se - cb
                for t in range(15):
                    word = jnp.where(t < kloc, word & (word - 1), word)
                bb = word & (-word)
                ee = (jnp.where((bb & 0xAAAA) != 0, 1, 0)
                      + jnp.where((bb & 0xCCCC) != 0, 2, 0)
                      + jnp.where((bb & 0xF0F0) != 0, 4, 0)
                      + jnp.where((bb & 0xFF00) != 0, 8, 0))
                fp = v * 16 + lane
                iq = fp >> log2ns
                row = q0 + iq
                fa = (row >> log2m) * n_pts
                val = pos * 16 + ee + fa
                obuf[pl.ds(v * 16, 16)] = jnp.where(tt > 0, val, fa)
                return 0

            lax.fori_loop(0, nv, fin, 0)
            pltpu.sync_copy(obuf, out.at[pl.ds(q0 * ns, lanes)])
            return 0

        lax.fori_loop(0, nbatches, batch, 0)

    return k(words_flat, c_flat)


def _mlp_pool_body(ns, cin, nlayers, h_ref, q_ref, *rest):
    w_refs = rest[:nlayers]
    b_refs = rest[nlayers:2 * nlayers]
    o_ref = rest[2 * nlayers]
    qb = h_ref.shape[0]
    hg = h_ref[...]
    q = q_ref[...]
    hx = hg[:, :, 0:3] - q[:, None, :]
    h = jnp.concatenate([hx, hg[:, :, 3:cin]], -1).reshape(qb * ns, cin)
    for i in range(nlayers):
        w = w_refs[i][...]
        b = b_refs[i][...]
        h = jnp.dot(h, w, preferred_element_type=jnp.float32) + b
        h = jnp.maximum(h, 0.0)
    c3 = h.shape[-1]
    o_ref[...] = jnp.max(h.reshape(qb, ns, c3), axis=1)


def _mlp_pool(h, q, cin, convs):
    rows, ns, dp = h.shape
    qb = 128
    nlayers = len(convs)
    wts = [jnp.transpose(c["W"]) for c in convs]
    bs = [c["b"][None, :] for c in convs]
    cout = convs[-1]["W"].shape[0]
    in_specs = [pl.BlockSpec((qb, ns, dp), lambda r: (r, 0, 0)),
                pl.BlockSpec((qb, 3), lambda r: (r, 0))]
    for w in wts:
        in_specs.append(pl.BlockSpec(w.shape, lambda r: (0, 0)))
    for b in bs:
        in_specs.append(pl.BlockSpec(b.shape, lambda r: (0, 0)))
    return pl.pallas_call(
        functools.partial(_mlp_pool_body, ns, cin, nlayers),
        grid=(rows // qb,),
        in_specs=in_specs,
        out_specs=pl.BlockSpec((qb, cout), lambda r: (r, 0)),
        out_shape=jax.ShapeDtypeStruct((rows, cout), jnp.float32),
    )(h, q, *wts, *bs)


def _matmul_relu_body(h_ref, w_ref, b_ref, o_ref):
    h = h_ref[...]
    o = jnp.dot(h, w_ref[...], preferred_element_type=jnp.float32) + b_ref[...]
    o_ref[...] = jnp.maximum(o, 0.0)


def _matmul_relu(h, w, b):
    rows, cin = h.shape
    qb = 256
    wt = jnp.transpose(w)
    cout = w.shape[0]
    return pl.pallas_call(
        _matmul_relu_body,
        grid=(rows // qb,),
        in_specs=[
            pl.BlockSpec((qb, cin), lambda r: (r, 0)),
            pl.BlockSpec(wt.shape, lambda r: (0, 0)),
            pl.BlockSpec((1, cout), lambda r: (0, 0)),
        ],
        out_specs=pl.BlockSpec((qb, cout), lambda r: (r, 0)),
        out_shape=jax.ShapeDtypeStruct((rows, cout), jnp.float32),
    )(h, wt, b[None, :])


def _sc_gather(table, idx):
    rt, d = table.shape
    r = idx.shape[0]
    mesh = plsc.VectorSubcoreMesh(core_axis_name="c", subcore_axis_name="s")
    info = plsc.get_sparse_core_info()
    nw = info.num_cores * info.num_subcores
    rpw = r // nw
    ch = min(128, rpw)
    nch = rpw // ch

    @functools.partial(
        pl.kernel, mesh=mesh,
        out_type=jax.ShapeDtypeStruct((r, d), jnp.float32),
        scratch_types=[
            pltpu.VMEM((1, ch), jnp.int32),
            pltpu.VMEM((ch, d), jnp.float32),
            pltpu.SemaphoreType.DMA,
        ],
        compiler_params=pltpu.CompilerParams(use_tc_tiling_on_sc=False),
    )
    def k(tab, ih, out, idxv, rows_v, sem):
        wid = lax.axis_index("s") * info.num_cores + lax.axis_index("c")
        base = wid * rpw

        def chunk(c, _):
            b0 = base + c * ch
            pltpu.sync_copy(ih.at[pl.ds(b0, ch)], idxv.at[0])
            pltpu.async_copy(tab.at[idxv.at[0]], rows_v, sem).wait()
            pltpu.sync_copy(rows_v, out.at[pl.ds(b0, ch)])
            return 0

        lax.fori_loop(0, nch, chunk, 0)

    return k(table, idx)


def _pad_table(x, dp):
    rt, d = x.shape
    if d == dp:
        return x
    return jnp.concatenate(
        [x, jnp.zeros((rt, dp - d), jnp.float32)], axis=-1)


def _topk_rank_body(jt, s_row_ref, s_col_ref, o_ref):
    b, ti, tj = pl.program_id(0), pl.program_id(1), pl.program_id(2)
    si = s_row_ref[0, :, :]
    sj = s_col_ref[0, :, :]
    ibase = ti * 128
    jbase = tj * jt
    iio = lax.broadcasted_iota(jnp.int32, (1, 128), 1) + ibase
    jio = lax.broadcasted_iota(jnp.int32, (jt, 1), 0) + jbase
    gt = (sj > si) | ((sj == si) & (jio < iio))
    cnt = jnp.sum(gt.astype(jnp.float32), axis=0, keepdims=True)
    prev = jnp.where(tj == 0, jnp.zeros((1, 128), jnp.float32), o_ref[0, :, :])
    o_ref[0, :, :] = prev + cnt


def _topk_sel_body(jt, m, rank_ref, o_ref):
    b, tp, tj = pl.program_id(0), pl.program_id(1), pl.program_id(2)
    rk = rank_ref[0, :, :]
    pio = lax.broadcasted_iota(jnp.int32, (1, 128), 1) + tp * 128
    ji = lax.broadcasted_iota(jnp.int32, (jt, 1), 0) + tj * jt + b * m
    hit = (rk == pio.astype(jnp.float32))
    contrib = jnp.sum(jnp.where(hit, ji.astype(jnp.float32), 0.0),
                      axis=0, keepdims=True)
    prev = jnp.where(tj == 0, jnp.zeros((1, 128), jnp.float32), o_ref[0, :, :])
    o_ref[0, :, :] = prev + contrib


def _topk_pallas(scores, k):
    b, m = scores.shape
    jt = 512
    s_row = scores[:, None, :]
    s_col = scores[:, :, None]
    rank = pl.pallas_call(
        functools.partial(_topk_rank_body, jt),
        grid=(b, m // 128, m // jt),
        in_specs=[
            pl.BlockSpec((1, 1, 128), lambda bb, i, j: (bb, 0, i)),
            pl.BlockSpec((1, jt, 1), lambda bb, i, j: (bb, j, 0)),
        ],
        out_specs=pl.BlockSpec((1, 1, 128), lambda bb, i, j: (bb, 0, i)),
        out_shape=jax.ShapeDtypeStruct((b, 1, m), jnp.float32),
    )(s_row, s_col)
    rank_col = jnp.transpose(rank, (0, 2, 1))
    sel = pl.pallas_call(
        functools.partial(_topk_sel_body, jt, m),
        grid=(b, k // 128, m // jt),
        in_specs=[
            pl.BlockSpec((1, jt, 1), lambda bb, p, j: (bb, j, 0)),
        ],
        out_specs=pl.BlockSpec((1, 1, 128), lambda bb, p, j: (bb, 0, p)),
        out_shape=jax.ShapeDtypeStruct((b, 1, k), jnp.float32),
    )(rank_col)
    return sel.reshape(b * k).astype(jnp.int32)


def _matmul_bias_body(h_ref, w_ref, b_ref, o_ref):
    h = h_ref[...]
    o_ref[...] = jnp.dot(h, w_ref[...],
                         preferred_element_type=jnp.float32) + b_ref[...]


def _matmul_bias(h, w, b):
    rows, cin = h.shape
    qb = 256
    wt = jnp.transpose(w)
    cout = w.shape[0]
    return pl.pallas_call(
        _matmul_bias_body,
        grid=(rows // qb,),
        in_specs=[
            pl.BlockSpec((qb, cin), lambda r: (r, 0)),
            pl.BlockSpec(wt.shape, lambda r: (0, 0)),
            pl.BlockSpec((1, cout), lambda r: (0, 0)),
        ],
        out_specs=pl.BlockSpec((qb, cout), lambda r: (r, 0)),
        out_shape=jax.ShapeDtypeStruct((rows, cout), jnp.float32),
    )(h, wt, b[None, :])


def _ball_query(dists, radius, nsample):
    n = dists.shape[-1]
    keyv = jnp.where(dists <= radius * radius,
                     jnp.arange(n, dtype=jnp.int32)[None, None, :], n)
    neg, _ = lax.top_k(-keyv, nsample)
    idx = -neg
    first = idx[:, :, :1]
    idx = jnp.where(idx == n, first, idx)
    idx = jnp.where(idx == n, 0, idx)
    return idx


def _identity_pallas(x):
    def body(x_ref, o_ref):
        o_ref[...] = x_ref[...]

    return pl.pallas_call(
        body,
        out_shape=jax.ShapeDtypeStruct(x.shape, x.dtype),
    )(x)


def _sa(points, feats_t, lp, li):
    npoint = _NUM_POINTS[li]
    b, n, _ = points.shape
    c = feats_t.shape[-1]
    cls_preds = None
    if _SAMPLING[li] == "ctr_aware":
        logits = _matmul_bias(feats_t.reshape(b * n, c),
                              lp["cls"]["W"], lp["cls"]["b"]).reshape(b, n, 3)
        scores = jnp.max(logits, -1)
        sel = _topk_pallas(scores, npoint)
        cls_preds = jnp.transpose(logits, (0, 2, 1))
        pts_pad = _pad_table(points.reshape(b * n, 3), 16)
        new_xyz = _sc_gather(pts_pad, sel)[:, :3].reshape(b, npoint, 3)
    else:
        new_xyz = _fps_pallas(lax.stop_gradient(points), npoint)
    pts_t3 = jnp.transpose(points, (0, 2, 1))
    words = _bq_masks(new_xyz, pts_t3, _RADII[li][0], _RADII[li][1])
    cin = 3 + c
    dp = -(-cin // 16) * 16
    tab = _pad_table(
        jnp.concatenate([points, feats_t], -1).reshape(b * n, cin), dp)
    q_flat = new_xyz.reshape(b * npoint, 3)
    outs = []
    for si, (r, ns) in enumerate(zip(_RADII[li], _NEIGHBORS[li])):
        flat = _bq_extract_sc(words[si], ns, n // 16, npoint, n)
        g = _sc_gather(tab, flat).reshape(b * npoint, ns, dp)
        pooled = _mlp_pool(g, q_flat, cin, lp["mlps"][si])
        outs.append(pooled)
    cat = jnp.concatenate(outs, -1)
    new_feat = _matmul_relu(cat, lp["agg"]["W"], lp["agg"]["b"])
    new_feat = new_feat.reshape(b, npoint, -1)
    return new_xyz, new_feat, cls_preds


def kernel(points, features, params):
    feats_t = jnp.transpose(features, (0, 2, 1))
    feats_t = _identity_pallas(feats_t)
    cls_list = []
    pts_list = []
    for li in range(3):
        ip = points
        points, feats_t, cp = _sa(points, feats_t, params["layers"][li], li)
        if cp is not None:
            cls_list.append(cp)
            pts_list.append(ip)
    return points, jnp.transpose(feats_t, (0, 2, 1)), cls_list, pts_list


# speedup vs baseline: 22.9675x; 1.0148x over previous
import functools

import jax
import jax.numpy as jnp
import jax.lax as lax
import numpy as np
from jax.experimental import pallas as pl
from jax.experimental.pallas import tpu as pltpu
from jax.experimental.pallas import tpu_sc as plsc

_B = 2
_N = 16384
_IN_C = 1
_NUM_CLASSES = 3
_NUM_POINTS = (4096, 1024, 512)
_SAMPLING = ("d-fps", "ctr_aware", "ctr_aware")
_NEIGHBORS = ((16, 32), (16, 32), (16, 32))
_RADII = ((0.2, 0.8), (0.8, 1.6), (1.6, 4.8))
_IN_LIST = (_IN_C, 64, 128)


def _gather_rows(x, idx):
    return jax.vmap(lambda xb, ib: xb[ib])(x, idx)


def _fps_body(npoint, n_rows, x_ref, o_ref):
    rows = n_rows
    iota_r = lax.broadcasted_iota(jnp.int32, (rows, 128), 0)
    iota_c = lax.broadcasted_iota(jnp.int32, (rows, 128), 1)
    gidx = iota_r * 128 + iota_c
    lane = lax.broadcasted_iota(jnp.int32, (1, 128), 1)

    x = x_ref[0, 0, :, :]
    y = x_ref[0, 1, :, :]
    z = x_ref[0, 2, :, :]

    def body(i, carry):
        dist, far = carry
        r = far // 128
        c = far % 128
        lmask = lane == c
        cx = jnp.sum(jnp.where(lmask, x_ref[0, 0, pl.ds(r, 1), :], 0.0))
        cy = jnp.sum(jnp.where(lmask, x_ref[0, 1, pl.ds(r, 1), :], 0.0))
        cz = jnp.sum(jnp.where(lmask, x_ref[0, 2, pl.ds(r, 1), :], 0.0))
        crow = jnp.where(lane == 0, cx,
                         jnp.where(lane == 1, cy,
                                   jnp.where(lane == 2, cz, 0.0)))
        o_ref[0, pl.ds(i, 1), :] = crow
        dx = x - cx
        dy = y - cy
        dz = z - cz
        d = dx * dx + dy * dy + dz * dz
        dist = jnp.minimum(dist, d)
        m = jnp.max(dist)
        far2 = jnp.min(jnp.where(dist == m, gidx, jnp.int32(1 << 30)))
        return dist, far2

    dist0 = jnp.full((rows, 128), 1e10, jnp.float32)
    lax.fori_loop(0, npoint, body, (dist0, jnp.int32(0)))


def _fps_pallas(points, npoint):
    b, n, _ = points.shape
    rows = n // 128
    pts = jnp.transpose(points, (0, 2, 1)).reshape(b, 3, rows, 128)
    out = pl.pallas_call(
        functools.partial(_fps_body, npoint, rows),
        grid=(b,),
        in_specs=[pl.BlockSpec((1, 3, rows, 128), lambda i: (i, 0, 0, 0))],
        out_specs=pl.BlockSpec((1, npoint, 128), lambda i: (i, 0, 0)),
        out_shape=jax.ShapeDtypeStruct((b, npoint, 128), jnp.float32),
    )(pts)
    return out[:, :, :3]


def _fps(xyz, npoint):
    b, n, _ = xyz.shape

    def body(i, state):
        cent, dist, far = state
        cent = cent.at[:, i].set(far)
        c = jnp.take_along_axis(xyz, far[:, None, None], axis=1)
        d = jnp.sum((xyz - c) ** 2, -1)
        dist = jnp.minimum(dist, d)
        far = jnp.argmax(dist, -1).astype(jnp.int32)
        return cent, dist, far

    cent = jnp.zeros((b, npoint), jnp.int32)
    dist = jnp.full((b, n), 1e10, jnp.float32)
    far = jnp.zeros((b,), jnp.int32)
    cent, _, _ = lax.fori_loop(0, npoint, body, (cent, dist, far))
    return cent


def _bq_mask_body(nc, r0sq, r1sq, nchunks, q_ref, p_ref,
                  w0_ref, w1_ref, c0_ref, c1_ref, cs_ref):
    wc = nc // 16
    qx = q_ref[0, :, 0:1]
    qy = q_ref[0, :, 1:2]
    qz = q_ref[0, :, 2:3]
    jidx = lax.broadcasted_iota(jnp.int32, (nc, wc), 0)
    widx = lax.broadcasted_iota(jnp.int32, (nc, wc), 1)
    blk = jidx // 16 == widx
    packm = jnp.where(blk, (jnp.int32(1) << (jidx % 16)), 0).astype(jnp.bfloat16)
    onesm = jnp.where(blk, 1, 0).astype(jnp.bfloat16)
    ta = lax.broadcasted_iota(jnp.int32, (wc, wc), 0)
    tb = lax.broadcasted_iota(jnp.int32, (wc, wc), 1)
    tri = jnp.where(ta <= tb, 1, 0).astype(jnp.bfloat16)

    cs_ref[:, :] = jnp.zeros((128, 2), jnp.float32)

    def chunk(k, _):
        base = pl.multiple_of(k * nc, nc)
        px = p_ref[0, 0:1, pl.ds(base, nc)]
        py = p_ref[0, 1:2, pl.ds(base, nc)]
        pz = p_ref[0, 2:3, pl.ds(base, nc)]
        dx = qx - px
        dy = qy - py
        dz = qz - pz
        d = dx * dx + dy * dy + dz * dz
        v0 = (d <= r0sq).astype(jnp.bfloat16)
        v1 = (d <= r1sq).astype(jnp.bfloat16)
        w0 = jax.lax.dot(v0, packm, preferred_element_type=jnp.float32)
        w1 = jax.lax.dot(v1, packm, preferred_element_type=jnp.float32)
        pc0 = jax.lax.dot(v0, onesm, preferred_element_type=jnp.float32)
        pc1 = jax.lax.dot(v1, onesm, preferred_element_type=jnp.float32)
        c0 = jax.lax.dot(pc0.astype(jnp.bfloat16), tri,
                         preferred_element_type=jnp.float32) + cs_ref[:, 0:1]
        c1 = jax.lax.dot(pc1.astype(jnp.bfloat16), tri,
                         preferred_element_type=jnp.float32) + cs_ref[:, 1:2]
        obase = pl.multiple_of(k * wc, wc)
        w0_ref[0, :, pl.ds(obase, wc)] = w0.astype(jnp.int32)
        w1_ref[0, :, pl.ds(obase, wc)] = w1.astype(jnp.int32)
        c0_ref[0, :, pl.ds(obase, wc)] = c0.astype(jnp.int32)
        c1_ref[0, :, pl.ds(obase, wc)] = c1.astype(jnp.int32)
        cs_ref[:, 0:1] = c0[:, wc - 1:wc]
        cs_ref[:, 1:2] = c1[:, wc - 1:wc]
        return 0

    lax.fori_loop(0, nchunks, chunk, 0)


def _bq_masks(new_xyz, pts_t, r0, r1):
    b, m, _ = new_xyz.shape
    n = pts_t.shape[-1]
    nc = min(n, 2048)
    nchunks = n // nc
    w = n // 16
    r0sq = np.float32(r0 * r0)
    r1sq = np.float32(r1 * r1)
    shp = jax.ShapeDtypeStruct((b, m, w), jnp.int32)
    out = pl.pallas_call(
        functools.partial(_bq_mask_body, nc, r0sq, r1sq, nchunks),
        grid=(b, m // 128),
        in_specs=[
            pl.BlockSpec((1, 128, 3), lambda i, j: (i, j, 0)),
            pl.BlockSpec((1, 3, n), lambda i, j: (i, 0, 0)),
        ],
        out_specs=[
            pl.BlockSpec((1, 128, w), lambda i, j: (i, j, 0)),
            pl.BlockSpec((1, 128, w), lambda i, j: (i, j, 0)),
            pl.BlockSpec((1, 128, w), lambda i, j: (i, j, 0)),
            pl.BlockSpec((1, 128, w), lambda i, j: (i, j, 0)),
        ],
        out_shape=[shp, shp, shp, shp],
        scratch_shapes=[pltpu.VMEM((128, 2), jnp.float32)],
    )(new_xyz, pts_t)
    flat = [x.reshape(b * m * w) for x in out]
    return (flat[0], flat[2]), (flat[1], flat[3])


def _bq_extract_sc(wc_pair, ns, w, m_per_batch, n_pts):
    words_flat, c_flat = wc_pair
    rows = words_flat.shape[0] // w
    mesh = plsc.VectorSubcoreMesh(core_axis_name="c", subcore_axis_name="s")
    info = plsc.get_sparse_core_info()
    nw = info.num_cores * info.num_subcores
    qpw = rows // nw
    lanes = min(1024, qpw * ns)
    nch = lanes // 128
    qb = lanes // ns
    nbatches = qpw // qb
    log2ns = ns.bit_length() - 1
    log2m = m_per_batch.bit_length() - 1
    strides = []
    st = w // 2
    while st >= 1:
        strides.append(st)
        st //= 2

    @functools.partial(
        pl.kernel, mesh=mesh,
        out_type=jax.ShapeDtypeStruct((rows * ns,), jnp.int32),
        scratch_types=[
            pltpu.VMEM((nch, 128), jnp.int32),
            pltpu.VMEM((nch, 128), jnp.int32),
            pltpu.VMEM((lanes,), jnp.int32),
            pltpu.VMEM((lanes,), jnp.int32),
            pltpu.VMEM((lanes,), jnp.int32),
            pltpu.VMEM((lanes,), jnp.int32),
            pltpu.VMEM((lanes,), jnp.int32),
            pltpu.SemaphoreType.DMA,
        ],
    )
    def k(wf, cf, out, idxb, gbuf, posb, cbb, seb, tbb, obuf, sem):
        wid = lax.axis_index("s") * info.num_cores + lax.axis_index("c")
        base_q = wid * qpw
        lane = lax.iota(jnp.int32, 16)
        nv = lanes // 16

        def gather_round(src):
            cps = [pltpu.async_copy(src.at[idxb.at[c]], gbuf.at[c], sem)
                   for c in range(nch)]
            for cp in cps:
                cp.wait()

        def batch(bi, _):
            q0 = base_q + bi * qb

            def init_idx(v, _):
                fp = v * 16 + lane
                iq = fp >> log2ns
                idxb[v // 8, pl.ds((v % 8) * 16, 16)] = (q0 + iq) * w + (w - 1)
                return 0

            lax.fori_loop(0, nv, init_idx, 0)
            gather_round(cf)

            def init2(v, _):
                fp = v * 16 + lane
                s = fp & (ns - 1)
                tt = gbuf[v // 8, pl.ds((v % 8) * 16, 16)]
                se = jnp.maximum(0, jnp.minimum(s, tt - 1))
                seb[pl.ds(v * 16, 16)] = se
                tbb[pl.ds(v * 16, 16)] = tt
                posb[pl.ds(v * 16, 16)] = jnp.zeros((16,), jnp.int32)
                cbb[pl.ds(v * 16, 16)] = jnp.zeros((16,), jnp.int32)
                return 0

            lax.fori_loop(0, nv, init2, 0)

            for stv in strides:
                def mkidx(v, _):
                    fp = v * 16 + lane
                    iq = fp >> log2ns
                    npos = posb[pl.ds(v * 16, 16)] + stv
                    idxb[v // 8, pl.ds((v % 8) * 16, 16)] = \
                        (q0 + iq) * w + npos - 1
                    return 0

                lax.fori_loop(0, nv, mkidx, 0)
                gather_round(cf)

                def upd(v, _):
                    cv = gbuf[v // 8, pl.ds((v % 8) * 16, 16)]
                    se = seb[pl.ds(v * 16, 16)]
                    pos = posb[pl.ds(v * 16, 16)]
                    ok = cv <= se
                    posb[pl.ds(v * 16, 16)] = jnp.where(ok, pos + stv, pos)
                    cb = cbb[pl.ds(v * 16, 16)]
                    cbb[pl.ds(v * 16, 16)] = jnp.where(ok, cv, cb)
                    return 0

                lax.fori_loop(0, nv, upd, 0)

            def widx(v, _):
                fp = v * 16 + lane
                iq = fp >> log2ns
                pos = posb[pl.ds(v * 16, 16)]
                idxb[v // 8, pl.ds((v % 8) * 16, 16)] = (q0 + iq) * w + pos
                return 0

            lax.fori_loop(0, nv, widx, 0)
            gather_round(wf)

            def fin(v, _):
                word = gbuf[v // 8, pl.ds((v % 8) * 16, 16)]
                se = seb[pl.ds(v * 16, 16)]
                cb = cbb[pl.ds(v * 16, 16)]
                pos = posb[pl.ds(v * 16, 16)]
                tt = tbb[pl.ds(v * 16, 16)]
                kloc = se - cb
                for t in range(15):
                    word = jnp.where(t < kloc, word & (word - 1), word)
                bb = word & (-word)
                ee = (jnp.where((bb & 0xAAAA) != 0, 1, 0)
                      + jnp.where((bb & 0xCCCC) != 0, 2, 0)
                      + jnp.where((bb & 0xF0F0) != 0, 4, 0)
                      + jnp.where((bb & 0xFF00) != 0, 8, 0))
                fp = v * 16 + lane
                iq = fp >> log2ns
                row = q0 + iq
                fa = (row >> log2m) * n_pts
                val = pos * 16 + ee + fa
                obuf[pl.ds(v * 16, 16)] = jnp.where(tt > 0, val, fa)
                return 0

            lax.fori_loop(0, nv, fin, 0)
            pltpu.sync_copy(obuf, out.at[pl.ds(q0 * ns, lanes)])
            return 0

        lax.fori_loop(0, nbatches, batch, 0)

    return k(words_flat, c_flat)


def _mlp_pool_body(ns, cin, nlayers, h_ref, q_ref, *rest):
    w_refs = rest[:nlayers]
    b_refs = rest[nlayers:2 * nlayers]
    o_ref = rest[2 * nlayers]
    qb = h_ref.shape[0]
    hg = h_ref[...]
    q = q_ref[...]
    hx = hg[:, :, 0:3] - q[:, None, :]
    h = jnp.concatenate([hx, hg[:, :, 3:cin]], -1).reshape(qb * ns, cin)
    for i in range(nlayers):
        w = w_refs[i][...]
        b = b_refs[i][...]
        h = jnp.dot(h, w, preferred_element_type=jnp.float32) + b
        h = jnp.maximum(h, 0.0)
    c3 = h.shape[-1]
    o_ref[...] = jnp.max(h.reshape(qb, ns, c3), axis=1)


def _mlp_pool(h, q, cin, convs):
    rows, ns, dp = h.shape
    qb = 128
    nlayers = len(convs)
    wts = [jnp.transpose(c["W"]) for c in convs]
    bs = [c["b"][None, :] for c in convs]
    cout = convs[-1]["W"].shape[0]
    in_specs = [pl.BlockSpec((qb, ns, dp), lambda r: (r, 0, 0)),
                pl.BlockSpec((qb, 3), lambda r: (r, 0))]
    for w in wts:
        in_specs.append(pl.BlockSpec(w.shape, lambda r: (0, 0)))
    for b in bs:
        in_specs.append(pl.BlockSpec(b.shape, lambda r: (0, 0)))
    return pl.pallas_call(
        functools.partial(_mlp_pool_body, ns, cin, nlayers),
        grid=(rows // qb,),
        in_specs=in_specs,
        out_specs=pl.BlockSpec((qb, cout), lambda r: (r, 0)),
        out_shape=jax.ShapeDtypeStruct((rows, cout), jnp.float32),
    )(h, q, *wts, *bs)


def _matmul_relu_body(h_ref, w_ref, b_ref, o_ref):
    h = h_ref[...]
    o = jnp.dot(h, w_ref[...], preferred_element_type=jnp.float32) + b_ref[...]
    o_ref[...] = jnp.maximum(o, 0.0)


def _matmul_relu(h, w, b):
    rows, cin = h.shape
    qb = 256
    wt = jnp.transpose(w)
    cout = w.shape[0]
    return pl.pallas_call(
        _matmul_relu_body,
        grid=(rows // qb,),
        in_specs=[
            pl.BlockSpec((qb, cin), lambda r: (r, 0)),
            pl.BlockSpec(wt.shape, lambda r: (0, 0)),
            pl.BlockSpec((1, cout), lambda r: (0, 0)),
        ],
        out_specs=pl.BlockSpec((qb, cout), lambda r: (r, 0)),
        out_shape=jax.ShapeDtypeStruct((rows, cout), jnp.float32),
    )(h, wt, b[None, :])


def _sc_gather(table, idx):
    rt, d = table.shape
    r = idx.shape[0]
    mesh = plsc.VectorSubcoreMesh(core_axis_name="c", subcore_axis_name="s")
    info = plsc.get_sparse_core_info()
    nw = info.num_cores * info.num_subcores
    rpw = r // nw
    ch = min(128, rpw)
    nch = rpw // ch

    @functools.partial(
        pl.kernel, mesh=mesh,
        out_type=jax.ShapeDtypeStruct((r, d), jnp.float32),
        scratch_types=[
            pltpu.VMEM((1, ch), jnp.int32),
            pltpu.VMEM((ch, d), jnp.float32),
            pltpu.SemaphoreType.DMA,
        ],
        compiler_params=pltpu.CompilerParams(use_tc_tiling_on_sc=False),
    )
    def k(tab, ih, out, idxv, rows_v, sem):
        wid = lax.axis_index("s") * info.num_cores + lax.axis_index("c")
        base = wid * rpw

        def chunk(c, _):
            b0 = base + c * ch
            pltpu.sync_copy(ih.at[pl.ds(b0, ch)], idxv.at[0])
            pltpu.async_copy(tab.at[idxv.at[0]], rows_v, sem).wait()
            pltpu.sync_copy(rows_v, out.at[pl.ds(b0, ch)])
            return 0

        lax.fori_loop(0, nch, chunk, 0)

    return k(table, idx)


def _pad_table(x, dp):
    rt, d = x.shape
    if d == dp:
        return x
    return jnp.concatenate(
        [x, jnp.zeros((rt, dp - d), jnp.float32)], axis=-1)


def _topk_rank_body(jt, s_row_ref, s_col_ref, o_ref):
    b, ti, tj = pl.program_id(0), pl.program_id(1), pl.program_id(2)
    si = s_row_ref[0, :, :]
    sj = s_col_ref[0, :, :]
    ibase = ti * 128
    jbase = tj * jt
    iio = lax.broadcasted_iota(jnp.int32, (1, 128), 1) + ibase
    jio = lax.broadcasted_iota(jnp.int32, (jt, 1), 0) + jbase
    gt = (sj > si) | ((sj == si) & (jio < iio))
    cnt = jnp.sum(gt.astype(jnp.float32), axis=0, keepdims=True)
    prev = jnp.where(tj == 0, jnp.zeros((1, 128), jnp.float32), o_ref[0, :, :])
    o_ref[0, :, :] = prev + cnt


def _topk_sel_body(jt, m, rank_ref, o_ref):
    b, tp, tj = pl.program_id(0), pl.program_id(1), pl.program_id(2)
    rk = rank_ref[0, :, :]
    pio = lax.broadcasted_iota(jnp.int32, (1, 128), 1) + tp * 128
    ji = lax.broadcasted_iota(jnp.int32, (jt, 1), 0) + tj * jt + b * m
    hit = (rk == pio.astype(jnp.float32))
    contrib = jnp.sum(jnp.where(hit, ji.astype(jnp.float32), 0.0),
                      axis=0, keepdims=True)
    prev = jnp.where(tj == 0, jnp.zeros((1, 128), jnp.float32), o_ref[0, :, :])
    o_ref[0, :, :] = prev + contrib


def _topk_pallas(scores, k):
    b, m = scores.shape
    jt = 512
    s_row = scores[:, None, :]
    s_col = scores[:, :, None]
    rank = pl.pallas_call(
        functools.partial(_topk_rank_body, jt),
        grid=(b, m // 128, m // jt),
        in_specs=[
            pl.BlockSpec((1, 1, 128), lambda bb, i, j: (bb, 0, i)),
            pl.BlockSpec((1, jt, 1), lambda bb, i, j: (bb, j, 0)),
        ],
        out_specs=pl.BlockSpec((1, 1, 128), lambda bb, i, j: (bb, 0, i)),
        out_shape=jax.ShapeDtypeStruct((b, 1, m), jnp.float32),
    )(s_row, s_col)
    rank_col = jnp.transpose(rank, (0, 2, 1))
    sel = pl.pallas_call(
        functools.partial(_topk_sel_body, jt, m),
        grid=(b, k // 128, m // jt),
        in_specs=[
            pl.BlockSpec((1, jt, 1), lambda bb, p, j: (bb, j, 0)),
        ],
        out_specs=pl.BlockSpec((1, 1, 128), lambda bb, p, j: (bb, 0, p)),
        out_shape=jax.ShapeDtypeStruct((b, 1, k), jnp.float32),
    )(rank_col)
    return sel.reshape(b * k).astype(jnp.int32)


def _matmul_bias_body(h_ref, w_ref, b_ref, o_ref):
    h = h_ref[...]
    o_ref[...] = jnp.dot(h, w_ref[...],
                         preferred_element_type=jnp.float32) + b_ref[...]


def _matmul_bias(h, w, b):
    rows, cin = h.shape
    qb = 256
    wt = jnp.transpose(w)
    cout = w.shape[0]
    return pl.pallas_call(
        _matmul_bias_body,
        grid=(rows // qb,),
        in_specs=[
            pl.BlockSpec((qb, cin), lambda r: (r, 0)),
            pl.BlockSpec(wt.shape, lambda r: (0, 0)),
            pl.BlockSpec((1, cout), lambda r: (0, 0)),
        ],
        out_specs=pl.BlockSpec((qb, cout), lambda r: (r, 0)),
        out_shape=jax.ShapeDtypeStruct((rows, cout), jnp.float32),
    )(h, wt, b[None, :])


def _ball_query(dists, radius, nsample):
    n = dists.shape[-1]
    keyv = jnp.where(dists <= radius * radius,
                     jnp.arange(n, dtype=jnp.int32)[None, None, :], n)
    neg, _ = lax.top_k(-keyv, nsample)
    idx = -neg
    first = idx[:, :, :1]
    idx = jnp.where(idx == n, first, idx)
    idx = jnp.where(idx == n, 0, idx)
    return idx


def _identity_pallas(x):
    def body(x_ref, o_ref):
        o_ref[...] = x_ref[...]

    return pl.pallas_call(
        body,
        out_shape=jax.ShapeDtypeStruct(x.shape, x.dtype),
    )(x)


def _sa(points, feats_t, lp, li):
    npoint = _NUM_POINTS[li]
    b, n, _ = points.shape
    c = feats_t.shape[-1]
    cls_preds = None
    if _SAMPLING[li] == "ctr_aware":
        logits = _matmul_bias(feats_t.reshape(b * n, c),
                              lp["cls"]["W"], lp["cls"]["b"]).reshape(b, n, 3)
        scores = jnp.max(logits, -1)
        sel = _topk_pallas(scores, npoint)
        cls_preds = jnp.transpose(logits, (0, 2, 1))
        pts_pad = _pad_table(points.reshape(b * n, 3), 16)
        new_xyz = _sc_gather(pts_pad, sel)[:, :3].reshape(b, npoint, 3)
    else:
        new_xyz = _fps_pallas(lax.stop_gradient(points), npoint)
    pts_t3 = jnp.transpose(points, (0, 2, 1))
    words = _bq_masks(new_xyz, pts_t3, _RADII[li][0], _RADII[li][1])
    cin = 3 + c
    dp = -(-cin // 16) * 16
    tab = _pad_table(
        jnp.concatenate([points, feats_t], -1).reshape(b * n, cin), dp)
    q_flat = new_xyz.reshape(b * npoint, 3)
    outs = []
    for si, (r, ns) in enumerate(zip(_RADII[li], _NEIGHBORS[li])):
        flat = _bq_extract_sc(words[si], ns, n // 16, npoint, n)
        g = _sc_gather(tab, flat).reshape(b * npoint, ns, dp)
        pooled = _mlp_pool(g, q_flat, cin, lp["mlps"][si])
        outs.append(pooled)
    cat = jnp.concatenate(outs, -1)
    new_feat = _matmul_relu(cat, lp["agg"]["W"], lp["agg"]["b"])
    new_feat = new_feat.reshape(b, npoint, -1)
    return new_xyz, new_feat, cls_preds


def kernel(points, features, params):
    feats_t = jnp.transpose(features, (0, 2, 1))
    feats_t = _identity_pallas(feats_t)
    cls_list = []
    pts_list = []
    for li in range(3):
        ip = points
        points, feats_t, cp = _sa(points, feats_t, params["layers"][li], li)
        if cp is not None:
            cls_list.append(cp)
            pts_list.append(ip)
    return points, jnp.transpose(feats_t, (0, 2, 1)), cls_list, pts_list


# speedup vs baseline: 24.9202x; 1.0850x over previous
import functools

import jax
import jax.numpy as jnp
import jax.lax as lax
import numpy as np
from jax.experimental import pallas as pl
from jax.experimental.pallas import tpu as pltpu
from jax.experimental.pallas import tpu_sc as plsc

_B = 2
_N = 16384
_IN_C = 1
_NUM_CLASSES = 3
_NUM_POINTS = (4096, 1024, 512)
_SAMPLING = ("d-fps", "ctr_aware", "ctr_aware")
_NEIGHBORS = ((16, 32), (16, 32), (16, 32))
_RADII = ((0.2, 0.8), (0.8, 1.6), (1.6, 4.8))
_IN_LIST = (_IN_C, 64, 128)


def _gather_rows(x, idx):
    return jax.vmap(lambda xb, ib: xb[ib])(x, idx)


def _fps_body(npoint, n_rows, x_ref, o_ref):
    rows = n_rows
    iota_r = lax.broadcasted_iota(jnp.int32, (rows, 128), 0)
    iota_c = lax.broadcasted_iota(jnp.int32, (rows, 128), 1)
    gidx = iota_r * 128 + iota_c
    lane = lax.broadcasted_iota(jnp.int32, (1, 128), 1)

    x = x_ref[0, 0, :, :]
    y = x_ref[0, 1, :, :]
    z = x_ref[0, 2, :, :]

    def body(i, carry):
        dist, far = carry
        r = far // 128
        c = far % 128
        lmask = lane == c
        cx = jnp.sum(jnp.where(lmask, x_ref[0, 0, pl.ds(r, 1), :], 0.0))
        cy = jnp.sum(jnp.where(lmask, x_ref[0, 1, pl.ds(r, 1), :], 0.0))
        cz = jnp.sum(jnp.where(lmask, x_ref[0, 2, pl.ds(r, 1), :], 0.0))
        crow = jnp.where(lane == 0, cx,
                         jnp.where(lane == 1, cy,
                                   jnp.where(lane == 2, cz, 0.0)))
        o_ref[0, pl.ds(i, 1), :] = crow
        dx = x - cx
        dy = y - cy
        dz = z - cz
        d = dx * dx + dy * dy + dz * dz
        dist = jnp.minimum(dist, d)
        m = jnp.max(dist)
        far2 = jnp.min(jnp.where(dist == m, gidx, jnp.int32(1 << 30)))
        return dist, far2

    dist0 = jnp.full((rows, 128), 1e10, jnp.float32)
    lax.fori_loop(0, npoint, body, (dist0, jnp.int32(0)))


def _fps_body2(npoint, n_rows, nb, x_ref, o_ref):
    rows = n_rows
    iota_r = lax.broadcasted_iota(jnp.int32, (rows, 128), 0)
    iota_c = lax.broadcasted_iota(jnp.int32, (rows, 128), 1)
    gidx = iota_r * 128 + iota_c
    lane = lax.broadcasted_iota(jnp.int32, (1, 128), 1)
    xs = [(x_ref[bi, 0, :, :], x_ref[bi, 1, :, :], x_ref[bi, 2, :, :])
          for bi in range(nb)]

    def body(i, carry):
        dists, fars = carry
        new_dists = []
        new_fars = []
        for bi in range(nb):
            dist = dists[bi]
            far = fars[bi]
            x, y, z = xs[bi]
            r = far // 128
            c = far % 128
            lmask = lane == c
            cx = jnp.sum(jnp.where(lmask, x_ref[bi, 0, pl.ds(r, 1), :], 0.0))
            cy = jnp.sum(jnp.where(lmask, x_ref[bi, 1, pl.ds(r, 1), :], 0.0))
            cz = jnp.sum(jnp.where(lmask, x_ref[bi, 2, pl.ds(r, 1), :], 0.0))
            crow = jnp.where(lane == 0, cx,
                             jnp.where(lane == 1, cy,
                                       jnp.where(lane == 2, cz, 0.0)))
            o_ref[bi, pl.ds(i, 1), :] = crow
            dx = x - cx
            dy = y - cy
            dz = z - cz
            d = dx * dx + dy * dy + dz * dz
            dist = jnp.minimum(dist, d)
            m = jnp.max(dist)
            far2 = jnp.min(jnp.where(dist == m, gidx, jnp.int32(1 << 30)))
            new_dists.append(dist)
            new_fars.append(far2)
        return tuple(new_dists), tuple(new_fars)

    dist0 = jnp.full((rows, 128), 1e10, jnp.float32)
    lax.fori_loop(0, npoint, body,
                  (tuple(dist0 for _ in range(nb)),
                   tuple(jnp.int32(0) for _ in range(nb))))


def _fps_pallas(points, npoint):
    b, n, _ = points.shape
    rows = n // 128
    pts = jnp.transpose(points, (0, 2, 1)).reshape(b, 3, rows, 128)
    out = pl.pallas_call(
        functools.partial(_fps_body2, npoint, rows, b),
        in_specs=[pl.BlockSpec((b, 3, rows, 128), lambda: (0, 0, 0, 0))],
        out_specs=pl.BlockSpec((b, npoint, 128), lambda: (0, 0, 0)),
        out_shape=jax.ShapeDtypeStruct((b, npoint, 128), jnp.float32),
    )(pts)
    return out[:, :, :3]


def _fps(xyz, npoint):
    b, n, _ = xyz.shape

    def body(i, state):
        cent, dist, far = state
        cent = cent.at[:, i].set(far)
        c = jnp.take_along_axis(xyz, far[:, None, None], axis=1)
        d = jnp.sum((xyz - c) ** 2, -1)
        dist = jnp.minimum(dist, d)
        far = jnp.argmax(dist, -1).astype(jnp.int32)
        return cent, dist, far

    cent = jnp.zeros((b, npoint), jnp.int32)
    dist = jnp.full((b, n), 1e10, jnp.float32)
    far = jnp.zeros((b,), jnp.int32)
    cent, _, _ = lax.fori_loop(0, npoint, body, (cent, dist, far))
    return cent


def _bq_mask_body(nc, r0sq, r1sq, nchunks, q_ref, p_ref,
                  w0_ref, w1_ref, c0_ref, c1_ref, cs_ref):
    wc = nc // 16
    qx = q_ref[0, :, 0:1]
    qy = q_ref[0, :, 1:2]
    qz = q_ref[0, :, 2:3]
    jidx = lax.broadcasted_iota(jnp.int32, (nc, wc), 0)
    widx = lax.broadcasted_iota(jnp.int32, (nc, wc), 1)
    blk = jidx // 16 == widx
    packm = jnp.where(blk, (jnp.int32(1) << (jidx % 16)), 0).astype(jnp.bfloat16)
    onesm = jnp.where(blk, 1, 0).astype(jnp.bfloat16)
    ta = lax.broadcasted_iota(jnp.int32, (wc, wc), 0)
    tb = lax.broadcasted_iota(jnp.int32, (wc, wc), 1)
    tri = jnp.where(ta <= tb, 1, 0).astype(jnp.bfloat16)

    cs_ref[:, :] = jnp.zeros((128, 2), jnp.float32)

    def chunk(k, _):
        base = pl.multiple_of(k * nc, nc)
        px = p_ref[0, 0:1, pl.ds(base, nc)]
        py = p_ref[0, 1:2, pl.ds(base, nc)]
        pz = p_ref[0, 2:3, pl.ds(base, nc)]
        dx = qx - px
        dy = qy - py
        dz = qz - pz
        d = dx * dx + dy * dy + dz * dz
        v0 = (d <= r0sq).astype(jnp.bfloat16)
        v1 = (d <= r1sq).astype(jnp.bfloat16)
        w0 = jax.lax.dot(v0, packm, preferred_element_type=jnp.float32)
        w1 = jax.lax.dot(v1, packm, preferred_element_type=jnp.float32)
        pc0 = jax.lax.dot(v0, onesm, preferred_element_type=jnp.float32)
        pc1 = jax.lax.dot(v1, onesm, preferred_element_type=jnp.float32)
        c0 = jax.lax.dot(pc0.astype(jnp.bfloat16), tri,
                         preferred_element_type=jnp.float32) + cs_ref[:, 0:1]
        c1 = jax.lax.dot(pc1.astype(jnp.bfloat16), tri,
                         preferred_element_type=jnp.float32) + cs_ref[:, 1:2]
        obase = pl.multiple_of(k * wc, wc)
        w0_ref[0, :, pl.ds(obase, wc)] = w0.astype(jnp.int32)
        w1_ref[0, :, pl.ds(obase, wc)] = w1.astype(jnp.int32)
        c0_ref[0, :, pl.ds(obase, wc)] = c0.astype(jnp.int32)
        c1_ref[0, :, pl.ds(obase, wc)] = c1.astype(jnp.int32)
        cs_ref[:, 0:1] = c0[:, wc - 1:wc]
        cs_ref[:, 1:2] = c1[:, wc - 1:wc]
        return 0

    lax.fori_loop(0, nchunks, chunk, 0)


def _bq_masks(new_xyz, pts_t, r0, r1):
    b, m, _ = new_xyz.shape
    n = pts_t.shape[-1]
    nc = min(n, 2048)
    nchunks = n // nc
    w = n // 16
    r0sq = np.float32(r0 * r0)
    r1sq = np.float32(r1 * r1)
    shp = jax.ShapeDtypeStruct((b, m, w), jnp.int32)
    out = pl.pallas_call(
        functools.partial(_bq_mask_body, nc, r0sq, r1sq, nchunks),
        grid=(b, m // 128),
        in_specs=[
            pl.BlockSpec((1, 128, 3), lambda i, j: (i, j, 0)),
            pl.BlockSpec((1, 3, n), lambda i, j: (i, 0, 0)),
        ],
        out_specs=[
            pl.BlockSpec((1, 128, w), lambda i, j: (i, j, 0)),
            pl.BlockSpec((1, 128, w), lambda i, j: (i, j, 0)),
            pl.BlockSpec((1, 128, w), lambda i, j: (i, j, 0)),
            pl.BlockSpec((1, 128, w), lambda i, j: (i, j, 0)),
        ],
        out_shape=[shp, shp, shp, shp],
        scratch_shapes=[pltpu.VMEM((128, 2), jnp.float32)],
    )(new_xyz, pts_t)
    flat = [x.reshape(b * m * w) for x in out]
    return (flat[0], flat[2]), (flat[1], flat[3])


def _bq_extract_sc(wc_pair, ns, w, m_per_batch, n_pts):
    words_flat, c_flat = wc_pair
    rows = words_flat.shape[0] // w
    mesh = plsc.VectorSubcoreMesh(core_axis_name="c", subcore_axis_name="s")
    info = plsc.get_sparse_core_info()
    nw = info.num_cores * info.num_subcores
    qpw = rows // nw
    lanes = min(1024, qpw * ns)
    nch = lanes // 128
    qb = lanes // ns
    nbatches = qpw // qb
    log2ns = ns.bit_length() - 1
    log2m = m_per_batch.bit_length() - 1
    strides = []
    st = w // 2
    while st >= 1:
        strides.append(st)
        st //= 2

    @functools.partial(
        pl.kernel, mesh=mesh,
        out_type=jax.ShapeDtypeStruct((rows * ns,), jnp.int32),
        scratch_types=[
            pltpu.VMEM((nch, 128), jnp.int32),
            pltpu.VMEM((nch, 128), jnp.int32),
            pltpu.VMEM((lanes,), jnp.int32),
            pltpu.VMEM((lanes,), jnp.int32),
            pltpu.VMEM((lanes,), jnp.int32),
            pltpu.VMEM((lanes,), jnp.int32),
            pltpu.VMEM((lanes,), jnp.int32),
            pltpu.SemaphoreType.DMA,
        ],
    )
    def k(wf, cf, out, idxb, gbuf, posb, cbb, seb, tbb, obuf, sem):
        wid = lax.axis_index("s") * info.num_cores + lax.axis_index("c")
        base_q = wid * qpw
        lane = lax.iota(jnp.int32, 16)
        nv = lanes // 16

        def gather_round(src):
            cps = [pltpu.async_copy(src.at[idxb.at[c]], gbuf.at[c], sem)
                   for c in range(nch)]
            for cp in cps:
                cp.wait()

        def batch(bi, _):
            q0 = base_q + bi * qb

            def init_idx(v, _):
                fp = v * 16 + lane
                iq = fp >> log2ns
                idxb[v // 8, pl.ds((v % 8) * 16, 16)] = (q0 + iq) * w + (w - 1)
                return 0

            lax.fori_loop(0, nv, init_idx, 0)
            gather_round(cf)

            def init2(v, _):
                fp = v * 16 + lane
                s = fp & (ns - 1)
                tt = gbuf[v // 8, pl.ds((v % 8) * 16, 16)]
                se = jnp.maximum(0, jnp.minimum(s, tt - 1))
                seb[pl.ds(v * 16, 16)] = se
                tbb[pl.ds(v * 16, 16)] = tt
                posb[pl.ds(v * 16, 16)] = jnp.zeros((16,), jnp.int32)
                cbb[pl.ds(v * 16, 16)] = jnp.zeros((16,), jnp.int32)
                return 0

            lax.fori_loop(0, nv, init2, 0)

            for stv in strides:
                def mkidx(v, _):
                    fp = v * 16 + lane
                    iq = fp >> log2ns
                    npos = posb[pl.ds(v * 16, 16)] + stv
                    idxb[v // 8, pl.ds((v % 8) * 16, 16)] = \
                        (q0 + iq) * w + npos - 1
                    return 0

                lax.fori_loop(0, nv, mkidx, 0)
                gather_round(cf)

                def upd(v, _):
                    cv = gbuf[v // 8, pl.ds((v % 8) * 16, 16)]
                    se = seb[pl.ds(v * 16, 16)]
                    pos = posb[pl.ds(v * 16, 16)]
                    ok = cv <= se
                    posb[pl.ds(v * 16, 16)] = jnp.where(ok, pos + stv, pos)
                    cb = cbb[pl.ds(v * 16, 16)]
                    cbb[pl.ds(v * 16, 16)] = jnp.where(ok, cv, cb)
                    return 0

                lax.fori_loop(0, nv, upd, 0)

            def widx(v, _):
                fp = v * 16 + lane
                iq = fp >> log2ns
                pos = posb[pl.ds(v * 16, 16)]
                idxb[v // 8, pl.ds((v % 8) * 16, 16)] = (q0 + iq) * w + pos
                return 0

            lax.fori_loop(0, nv, widx, 0)
            gather_round(wf)

            def fin(v, _):
                word = gbuf[v // 8, pl.ds((v % 8) * 16, 16)]
                se = seb[pl.ds(v * 16, 16)]
                cb = cbb[pl.ds(v * 16, 16)]
                pos = posb[pl.ds(v * 16, 16)]
                tt = tbb[pl.ds(v * 16, 16)]
                kloc = se - cb
                for t in range(15):
                    word = jnp.where(t < kloc, word & (word - 1), word)
                bb = word & (-word)
                ee = (jnp.where((bb & 0xAAAA) != 0, 1, 0)
                      + jnp.where((bb & 0xCCCC) != 0, 2, 0)
                      + jnp.where((bb & 0xF0F0) != 0, 4, 0)
                      + jnp.where((bb & 0xFF00) != 0, 8, 0))
                fp = v * 16 + lane
                iq = fp >> log2ns
                row = q0 + iq
                fa = (row >> log2m) * n_pts
                val = pos * 16 + ee + fa
                obuf[pl.ds(v * 16, 16)] = jnp.where(tt > 0, val, fa)
                return 0

            lax.fori_loop(0, nv, fin, 0)
            pltpu.sync_copy(obuf, out.at[pl.ds(q0 * ns, lanes)])
            return 0

        lax.fori_loop(0, nbatches, batch, 0)

    return k(words_flat, c_flat)


def _mlp_pool_body(ns, cin, nlayers, h_ref, q_ref, *rest):
    w_refs = rest[:nlayers]
    b_refs = rest[nlayers:2 * nlayers]
    o_ref = rest[2 * nlayers]
    qb = h_ref.shape[0]
    hg = h_ref[...]
    q = q_ref[...]
    hx = hg[:, :, 0:3] - q[:, None, :]
    h = jnp.concatenate([hx, hg[:, :, 3:cin]], -1).reshape(qb * ns, cin)
    for i in range(nlayers):
        w = w_refs[i][...]
        b = b_refs[i][...]
        h = jnp.dot(h, w, preferred_element_type=jnp.float32) + b
        h = jnp.maximum(h, 0.0)
    c3 = h.shape[-1]
    o_ref[...] = jnp.max(h.reshape(qb, ns, c3), axis=1)


def _mlp_pool(h, q, cin, convs):
    rows, ns, dp = h.shape
    qb = 128
    nlayers = len(convs)
    wts = [jnp.transpose(c["W"]) for c in convs]
    bs = [c["b"][None, :] for c in convs]
    cout = convs[-1]["W"].shape[0]
    in_specs = [pl.BlockSpec((qb, ns, dp), lambda r: (r, 0, 0)),
                pl.BlockSpec((qb, 3), lambda r: (r, 0))]
    for w in wts:
        in_specs.append(pl.BlockSpec(w.shape, lambda r: (0, 0)))
    for b in bs:
        in_specs.append(pl.BlockSpec(b.shape, lambda r: (0, 0)))
    return pl.pallas_call(
        functools.partial(_mlp_pool_body, ns, cin, nlayers),
        grid=(rows // qb,),
        in_specs=in_specs,
        out_specs=pl.BlockSpec((qb, cout), lambda r: (r, 0)),
        out_shape=jax.ShapeDtypeStruct((rows, cout), jnp.float32),
    )(h, q, *wts, *bs)


def _matmul_relu_body(h_ref, w_ref, b_ref, o_ref):
    h = h_ref[...]
    o = jnp.dot(h, w_ref[...], preferred_element_type=jnp.float32) + b_ref[...]
    o_ref[...] = jnp.maximum(o, 0.0)


def _matmul_relu(h, w, b):
    rows, cin = h.shape
    qb = 256
    wt = jnp.transpose(w)
    cout = w.shape[0]
    return pl.pallas_call(
        _matmul_relu_body,
        grid=(rows // qb,),
        in_specs=[
            pl.BlockSpec((qb, cin), lambda r: (r, 0)),
            pl.BlockSpec(wt.shape, lambda r: (0, 0)),
            pl.BlockSpec((1, cout), lambda r: (0, 0)),
        ],
        out_specs=pl.BlockSpec((qb, cout), lambda r: (r, 0)),
        out_shape=jax.ShapeDtypeStruct((rows, cout), jnp.float32),
    )(h, wt, b[None, :])


def _sc_gather(table, idx):
    rt, d = table.shape
    r = idx.shape[0]
    mesh = plsc.VectorSubcoreMesh(core_axis_name="c", subcore_axis_name="s")
    info = plsc.get_sparse_core_info()
    nw = info.num_cores * info.num_subcores
    rpw = r // nw
    ch = min(128, rpw)
    nch = rpw // ch

    @functools.partial(
        pl.kernel, mesh=mesh,
        out_type=jax.ShapeDtypeStruct((r, d), jnp.float32),
        scratch_types=[
            pltpu.VMEM((1, ch), jnp.int32),
            pltpu.VMEM((ch, d), jnp.float32),
            pltpu.SemaphoreType.DMA,
        ],
        compiler_params=pltpu.CompilerParams(use_tc_tiling_on_sc=False),
    )
    def k(tab, ih, out, idxv, rows_v, sem):
        wid = lax.axis_index("s") * info.num_cores + lax.axis_index("c")
        base = wid * rpw

        def chunk(c, _):
            b0 = base + c * ch
            pltpu.sync_copy(ih.at[pl.ds(b0, ch)], idxv.at[0])
            pltpu.async_copy(tab.at[idxv.at[0]], rows_v, sem).wait()
            pltpu.sync_copy(rows_v, out.at[pl.ds(b0, ch)])
            return 0

        lax.fori_loop(0, nch, chunk, 0)

    return k(table, idx)


def _pad_table(x, dp):
    rt, d = x.shape
    if d == dp:
        return x
    return jnp.concatenate(
        [x, jnp.zeros((rt, dp - d), jnp.float32)], axis=-1)


def _topk_rank_body(jt, s_row_ref, s_col_ref, o_ref):
    b, ti, tj = pl.program_id(0), pl.program_id(1), pl.program_id(2)
    si = s_row_ref[0, :, :]
    sj = s_col_ref[0, :, :]
    ibase = ti * 128
    jbase = tj * jt
    iio = lax.broadcasted_iota(jnp.int32, (1, 128), 1) + ibase
    jio = lax.broadcasted_iota(jnp.int32, (jt, 1), 0) + jbase
    gt = (sj > si) | ((sj == si) & (jio < iio))
    cnt = jnp.sum(gt.astype(jnp.float32), axis=0, keepdims=True)
    prev = jnp.where(tj == 0, jnp.zeros((1, 128), jnp.float32), o_ref[0, :, :])
    o_ref[0, :, :] = prev + cnt


def _topk_sel_body(jt, m, rank_ref, o_ref):
    b, tp, tj = pl.program_id(0), pl.program_id(1), pl.program_id(2)
    rk = rank_ref[0, :, :]
    pio = lax.broadcasted_iota(jnp.int32, (1, 128), 1) + tp * 128
    ji = lax.broadcasted_iota(jnp.int32, (jt, 1), 0) + tj * jt + b * m
    hit = (rk == pio.astype(jnp.float32))
    contrib = jnp.sum(jnp.where(hit, ji.astype(jnp.float32), 0.0),
                      axis=0, keepdims=True)
    prev = jnp.where(tj == 0, jnp.zeros((1, 128), jnp.float32), o_ref[0, :, :])
    o_ref[0, :, :] = prev + contrib


def _topk_pallas(scores, k):
    b, m = scores.shape
    jt = 512
    s_row = scores[:, None, :]
    s_col = scores[:, :, None]
    rank = pl.pallas_call(
        functools.partial(_topk_rank_body, jt),
        grid=(b, m // 128, m // jt),
        in_specs=[
            pl.BlockSpec((1, 1, 128), lambda bb, i, j: (bb, 0, i)),
            pl.BlockSpec((1, jt, 1), lambda bb, i, j: (bb, j, 0)),
        ],
        out_specs=pl.BlockSpec((1, 1, 128), lambda bb, i, j: (bb, 0, i)),
        out_shape=jax.ShapeDtypeStruct((b, 1, m), jnp.float32),
    )(s_row, s_col)
    rank_col = jnp.transpose(rank, (0, 2, 1))
    sel = pl.pallas_call(
        functools.partial(_topk_sel_body, jt, m),
        grid=(b, k // 128, m // jt),
        in_specs=[
            pl.BlockSpec((1, jt, 1), lambda bb, p, j: (bb, j, 0)),
        ],
        out_specs=pl.BlockSpec((1, 1, 128), lambda bb, p, j: (bb, 0, p)),
        out_shape=jax.ShapeDtypeStruct((b, 1, k), jnp.float32),
    )(rank_col)
    return sel.reshape(b * k).astype(jnp.int32)


def _matmul_bias_body(h_ref, w_ref, b_ref, o_ref):
    h = h_ref[...]
    o_ref[...] = jnp.dot(h, w_ref[...],
                         preferred_element_type=jnp.float32) + b_ref[...]


def _matmul_bias(h, w, b):
    rows, cin = h.shape
    qb = 256
    wt = jnp.transpose(w)
    cout = w.shape[0]
    return pl.pallas_call(
        _matmul_bias_body,
        grid=(rows // qb,),
        in_specs=[
            pl.BlockSpec((qb, cin), lambda r: (r, 0)),
            pl.BlockSpec(wt.shape, lambda r: (0, 0)),
            pl.BlockSpec((1, cout), lambda r: (0, 0)),
        ],
        out_specs=pl.BlockSpec((qb, cout), lambda r: (r, 0)),
        out_shape=jax.ShapeDtypeStruct((rows, cout), jnp.float32),
    )(h, wt, b[None, :])


def _ball_query(dists, radius, nsample):
    n = dists.shape[-1]
    keyv = jnp.where(dists <= radius * radius,
                     jnp.arange(n, dtype=jnp.int32)[None, None, :], n)
    neg, _ = lax.top_k(-keyv, nsample)
    idx = -neg
    first = idx[:, :, :1]
    idx = jnp.where(idx == n, first, idx)
    idx = jnp.where(idx == n, 0, idx)
    return idx


def _identity_pallas(x):
    def body(x_ref, o_ref):
        o_ref[...] = x_ref[...]

    return pl.pallas_call(
        body,
        out_shape=jax.ShapeDtypeStruct(x.shape, x.dtype),
    )(x)


def _sa(points, feats_t, lp, li):
    npoint = _NUM_POINTS[li]
    b, n, _ = points.shape
    c = feats_t.shape[-1]
    cls_preds = None
    if _SAMPLING[li] == "ctr_aware":
        logits = _matmul_bias(feats_t.reshape(b * n, c),
                              lp["cls"]["W"], lp["cls"]["b"]).reshape(b, n, 3)
        scores = jnp.max(logits, -1)
        sel = _topk_pallas(scores, npoint)
        cls_preds = jnp.transpose(logits, (0, 2, 1))
        pts_pad = _pad_table(points.reshape(b * n, 3), 16)
        new_xyz = _sc_gather(pts_pad, sel)[:, :3].reshape(b, npoint, 3)
    else:
        new_xyz = _fps_pallas(lax.stop_gradient(points), npoint)
    pts_t3 = jnp.transpose(points, (0, 2, 1))
    words = _bq_masks(new_xyz, pts_t3, _RADII[li][0], _RADII[li][1])
    cin = 3 + c
    dp = -(-cin // 16) * 16
    tab = _pad_table(
        jnp.concatenate([points, feats_t], -1).reshape(b * n, cin), dp)
    q_flat = new_xyz.reshape(b * npoint, 3)
    outs = []
    for si, (r, ns) in enumerate(zip(_RADII[li], _NEIGHBORS[li])):
        flat = _bq_extract_sc(words[si], ns, n // 16, npoint, n)
        g = _sc_gather(tab, flat).reshape(b * npoint, ns, dp)
        pooled = _mlp_pool(g, q_flat, cin, lp["mlps"][si])
        outs.append(pooled)
    cat = jnp.concatenate(outs, -1)
    new_feat = _matmul_relu(cat, lp["agg"]["W"], lp["agg"]["b"])
    new_feat = new_feat.reshape(b, npoint, -1)
    return new_xyz, new_feat, cls_preds


def kernel(points, features, params):
    feats_t = jnp.transpose(features, (0, 2, 1))
    feats_t = _identity_pallas(feats_t)
    cls_list = []
    pts_list = []
    for li in range(3):
        ip = points
        points, feats_t, cp = _sa(points, feats_t, params["layers"][li], li)
        if cp is not None:
            cls_list.append(cp)
            pts_list.append(ip)
    return points, jnp.transpose(feats_t, (0, 2, 1)), cls_list, pts_list


# speedup vs baseline: 25.4746x; 1.0222x over previous
import functools

import jax
import jax.numpy as jnp
import jax.lax as lax
import numpy as np
from jax.experimental import pallas as pl
from jax.experimental.pallas import tpu as pltpu
from jax.experimental.pallas import tpu_sc as plsc

_B = 2
_N = 16384
_IN_C = 1
_NUM_CLASSES = 3
_NUM_POINTS = (4096, 1024, 512)
_SAMPLING = ("d-fps", "ctr_aware", "ctr_aware")
_NEIGHBORS = ((16, 32), (16, 32), (16, 32))
_RADII = ((0.2, 0.8), (0.8, 1.6), (1.6, 4.8))
_IN_LIST = (_IN_C, 64, 128)


def _gather_rows(x, idx):
    return jax.vmap(lambda xb, ib: xb[ib])(x, idx)


def _fps_body(npoint, n_rows, x_ref, o_ref):
    rows = n_rows
    iota_r = lax.broadcasted_iota(jnp.int32, (rows, 128), 0)
    iota_c = lax.broadcasted_iota(jnp.int32, (rows, 128), 1)
    gidx = iota_r * 128 + iota_c
    lane = lax.broadcasted_iota(jnp.int32, (1, 128), 1)

    x = x_ref[0, 0, :, :]
    y = x_ref[0, 1, :, :]
    z = x_ref[0, 2, :, :]

    def body(i, carry):
        dist, far = carry
        r = far // 128
        c = far % 128
        lmask = lane == c
        cx = jnp.sum(jnp.where(lmask, x_ref[0, 0, pl.ds(r, 1), :], 0.0))
        cy = jnp.sum(jnp.where(lmask, x_ref[0, 1, pl.ds(r, 1), :], 0.0))
        cz = jnp.sum(jnp.where(lmask, x_ref[0, 2, pl.ds(r, 1), :], 0.0))
        crow = jnp.where(lane == 0, cx,
                         jnp.where(lane == 1, cy,
                                   jnp.where(lane == 2, cz, 0.0)))
        o_ref[0, pl.ds(i, 1), :] = crow
        dx = x - cx
        dy = y - cy
        dz = z - cz
        d = dx * dx + dy * dy + dz * dz
        dist = jnp.minimum(dist, d)
        m = jnp.max(dist)
        far2 = jnp.min(jnp.where(dist == m, gidx, jnp.int32(1 << 30)))
        return dist, far2

    dist0 = jnp.full((rows, 128), 1e10, jnp.float32)
    lax.fori_loop(0, npoint, body, (dist0, jnp.int32(0)))


def _fps_body2(npoint, n_rows, nb, x_ref, o_ref):
    rows = n_rows
    iota_r = lax.broadcasted_iota(jnp.int32, (rows, 128), 0)
    iota_c = lax.broadcasted_iota(jnp.int32, (rows, 128), 1)
    gidx = iota_r * 128 + iota_c
    lane = lax.broadcasted_iota(jnp.int32, (1, 128), 1)
    xs = [(x_ref[bi, 0, :, :], x_ref[bi, 1, :, :], x_ref[bi, 2, :, :])
          for bi in range(nb)]

    def body(i, carry):
        dists, fars = carry
        new_dists = []
        new_fars = []
        for bi in range(nb):
            dist = dists[bi]
            far = fars[bi]
            x, y, z = xs[bi]
            r = far // 128
            c = far % 128
            lmask = lane == c
            cx = jnp.sum(jnp.where(lmask, x_ref[bi, 0, pl.ds(r, 1), :], 0.0))
            cy = jnp.sum(jnp.where(lmask, x_ref[bi, 1, pl.ds(r, 1), :], 0.0))
            cz = jnp.sum(jnp.where(lmask, x_ref[bi, 2, pl.ds(r, 1), :], 0.0))
            crow = jnp.where(lane == 0, cx,
                             jnp.where(lane == 1, cy,
                                       jnp.where(lane == 2, cz, 0.0)))
            o_ref[bi, pl.ds(i, 1), :] = crow
            dx = x - cx
            dy = y - cy
            dz = z - cz
            d = dx * dx + dy * dy + dz * dz
            dist = jnp.minimum(dist, d)
            m = jnp.max(dist)
            far2 = jnp.min(jnp.where(dist == m, gidx, jnp.int32(1 << 30)))
            new_dists.append(dist)
            new_fars.append(far2)
        return tuple(new_dists), tuple(new_fars)

    dist0 = jnp.full((rows, 128), 1e10, jnp.float32)
    lax.fori_loop(0, npoint, body,
                  (tuple(dist0 for _ in range(nb)),
                   tuple(jnp.int32(0) for _ in range(nb))))


def _fps_pallas(points, npoint):
    b, n, _ = points.shape
    rows = n // 128
    pts = jnp.transpose(points, (0, 2, 1)).reshape(b, 3, rows, 128)
    out = pl.pallas_call(
        functools.partial(_fps_body2, npoint, rows, b),
        in_specs=[pl.BlockSpec((b, 3, rows, 128), lambda: (0, 0, 0, 0))],
        out_specs=pl.BlockSpec((b, npoint, 128), lambda: (0, 0, 0)),
        out_shape=jax.ShapeDtypeStruct((b, npoint, 128), jnp.float32),
    )(pts)
    return out[:, :, :3]


def _fps(xyz, npoint):
    b, n, _ = xyz.shape

    def body(i, state):
        cent, dist, far = state
        cent = cent.at[:, i].set(far)
        c = jnp.take_along_axis(xyz, far[:, None, None], axis=1)
        d = jnp.sum((xyz - c) ** 2, -1)
        dist = jnp.minimum(dist, d)
        far = jnp.argmax(dist, -1).astype(jnp.int32)
        return cent, dist, far

    cent = jnp.zeros((b, npoint), jnp.int32)
    dist = jnp.full((b, n), 1e10, jnp.float32)
    far = jnp.zeros((b,), jnp.int32)
    cent, _, _ = lax.fori_loop(0, npoint, body, (cent, dist, far))
    return cent


def _bq_mask_body(nc, r0sq, r1sq, nchunks, q_ref, p_ref,
                  w0_ref, w1_ref, c0_ref, c1_ref, cs_ref):
    wc = nc // 16
    qx = q_ref[0, :, 0:1]
    qy = q_ref[0, :, 1:2]
    qz = q_ref[0, :, 2:3]
    jidx = lax.broadcasted_iota(jnp.int32, (nc, wc), 0)
    widx = lax.broadcasted_iota(jnp.int32, (nc, wc), 1)
    blk = jidx // 16 == widx
    packm = jnp.where(blk, (jnp.int32(1) << (jidx % 16)), 0).astype(jnp.bfloat16)
    onesm = jnp.where(blk, 1, 0).astype(jnp.bfloat16)
    ta = lax.broadcasted_iota(jnp.int32, (wc, wc), 0)
    tb = lax.broadcasted_iota(jnp.int32, (wc, wc), 1)
    tri = jnp.where(ta <= tb, 1, 0).astype(jnp.bfloat16)

    cs_ref[:, :] = jnp.zeros((128, 2), jnp.float32)

    def chunk(k, _):
        base = pl.multiple_of(k * nc, nc)
        px = p_ref[0, 0:1, pl.ds(base, nc)]
        py = p_ref[0, 1:2, pl.ds(base, nc)]
        pz = p_ref[0, 2:3, pl.ds(base, nc)]
        dx = qx - px
        dy = qy - py
        dz = qz - pz
        d = dx * dx + dy * dy + dz * dz
        v0 = (d <= r0sq).astype(jnp.bfloat16)
        v1 = (d <= r1sq).astype(jnp.bfloat16)
        w0 = jax.lax.dot(v0, packm, preferred_element_type=jnp.float32)
        w1 = jax.lax.dot(v1, packm, preferred_element_type=jnp.float32)
        pc0 = jax.lax.dot(v0, onesm, preferred_element_type=jnp.float32)
        pc1 = jax.lax.dot(v1, onesm, preferred_element_type=jnp.float32)
        c0 = jax.lax.dot(pc0.astype(jnp.bfloat16), tri,
                         preferred_element_type=jnp.float32) + cs_ref[:, 0:1]
        c1 = jax.lax.dot(pc1.astype(jnp.bfloat16), tri,
                         preferred_element_type=jnp.float32) + cs_ref[:, 1:2]
        obase = pl.multiple_of(k * wc, wc)
        w0_ref[0, :, pl.ds(obase, wc)] = w0.astype(jnp.int32)
        w1_ref[0, :, pl.ds(obase, wc)] = w1.astype(jnp.int32)
        c0_ref[0, :, pl.ds(obase, wc)] = c0.astype(jnp.int32)
        c1_ref[0, :, pl.ds(obase, wc)] = c1.astype(jnp.int32)
        cs_ref[:, 0:1] = c0[:, wc - 1:wc]
        cs_ref[:, 1:2] = c1[:, wc - 1:wc]
        return 0

    lax.fori_loop(0, nchunks, chunk, 0)


def _bq_masks(new_xyz, pts_t, r0, r1):
    b, m, _ = new_xyz.shape
    n = pts_t.shape[-1]
    nc = min(n, 2048)
    nchunks = n // nc
    w = n // 16
    r0sq = np.float32(r0 * r0)
    r1sq = np.float32(r1 * r1)
    shp = jax.ShapeDtypeStruct((b, m, w), jnp.int32)
    out = pl.pallas_call(
        functools.partial(_bq_mask_body, nc, r0sq, r1sq, nchunks),
        grid=(b, m // 128),
        in_specs=[
            pl.BlockSpec((1, 128, 3), lambda i, j: (i, j, 0)),
            pl.BlockSpec((1, 3, n), lambda i, j: (i, 0, 0)),
        ],
        out_specs=[
            pl.BlockSpec((1, 128, w), lambda i, j: (i, j, 0)),
            pl.BlockSpec((1, 128, w), lambda i, j: (i, j, 0)),
            pl.BlockSpec((1, 128, w), lambda i, j: (i, j, 0)),
            pl.BlockSpec((1, 128, w), lambda i, j: (i, j, 0)),
        ],
        out_shape=[shp, shp, shp, shp],
        scratch_shapes=[pltpu.VMEM((128, 2), jnp.float32)],
    )(new_xyz, pts_t)
    flat = [x.reshape(b * m * w) for x in out]
    return (flat[0], flat[2]), (flat[1], flat[3])


def _bq_extract_sc(wc_pair, ns, w, m_per_batch, n_pts):
    words_flat, c_flat = wc_pair
    rows = words_flat.shape[0] // w
    mesh = plsc.VectorSubcoreMesh(core_axis_name="c", subcore_axis_name="s")
    info = plsc.get_sparse_core_info()
    nw = info.num_cores * info.num_subcores
    qpw = rows // nw
    lanes = min(1024, qpw * ns)
    nch = lanes // 128
    qb = lanes // ns
    nbatches = qpw // qb
    log2ns = ns.bit_length() - 1
    log2m = m_per_batch.bit_length() - 1
    strides = []
    st = w // 2
    while st >= 1:
        strides.append(st)
        st //= 2

    @functools.partial(
        pl.kernel, mesh=mesh,
        out_type=jax.ShapeDtypeStruct((rows * ns,), jnp.int32),
        scratch_types=[
            pltpu.VMEM((nch, 128), jnp.int32),
            pltpu.VMEM((nch, 128), jnp.int32),
            pltpu.VMEM((lanes,), jnp.int32),
            pltpu.VMEM((lanes,), jnp.int32),
            pltpu.VMEM((lanes,), jnp.int32),
            pltpu.VMEM((lanes,), jnp.int32),
            pltpu.VMEM((lanes,), jnp.int32),
            pltpu.SemaphoreType.DMA,
        ],
    )
    def k(wf, cf, out, idxb, gbuf, posb, cbb, seb, tbb, obuf, sem):
        wid = lax.axis_index("s") * info.num_cores + lax.axis_index("c")
        base_q = wid * qpw
        lane = lax.iota(jnp.int32, 16)
        nv = lanes // 16

        def gather_round(src):
            cps = [pltpu.async_copy(src.at[idxb.at[c]], gbuf.at[c], sem)
                   for c in range(nch)]
            for cp in cps:
                cp.wait()

        def batch(bi, _):
            q0 = base_q + bi * qb

            def init_idx(v, _):
                fp = v * 16 + lane
                iq = fp >> log2ns
                idxb[v // 8, pl.ds((v % 8) * 16, 16)] = (q0 + iq) * w + (w - 1)
                return 0

            lax.fori_loop(0, nv, init_idx, 0)
            gather_round(cf)

            def init2(v, _):
                fp = v * 16 + lane
                s = fp & (ns - 1)
                tt = gbuf[v // 8, pl.ds((v % 8) * 16, 16)]
                se = jnp.maximum(0, jnp.minimum(s, tt - 1))
                seb[pl.ds(v * 16, 16)] = se
                tbb[pl.ds(v * 16, 16)] = tt
                posb[pl.ds(v * 16, 16)] = jnp.zeros((16,), jnp.int32)
                cbb[pl.ds(v * 16, 16)] = jnp.zeros((16,), jnp.int32)
                return 0

            lax.fori_loop(0, nv, init2, 0)

            for stv in strides:
                def mkidx(v, _):
                    fp = v * 16 + lane
                    iq = fp >> log2ns
                    npos = posb[pl.ds(v * 16, 16)] + stv
                    idxb[v // 8, pl.ds((v % 8) * 16, 16)] = \
                        (q0 + iq) * w + npos - 1
                    return 0

                lax.fori_loop(0, nv, mkidx, 0)
                gather_round(cf)

                def upd(v, _):
                    cv = gbuf[v // 8, pl.ds((v % 8) * 16, 16)]
                    se = seb[pl.ds(v * 16, 16)]
                    pos = posb[pl.ds(v * 16, 16)]
                    ok = cv <= se
                    posb[pl.ds(v * 16, 16)] = jnp.where(ok, pos + stv, pos)
                    cb = cbb[pl.ds(v * 16, 16)]
                    cbb[pl.ds(v * 16, 16)] = jnp.where(ok, cv, cb)
                    return 0

                lax.fori_loop(0, nv, upd, 0)

            def widx(v, _):
                fp = v * 16 + lane
                iq = fp >> log2ns
                pos = posb[pl.ds(v * 16, 16)]
                idxb[v // 8, pl.ds((v % 8) * 16, 16)] = (q0 + iq) * w + pos
                return 0

            lax.fori_loop(0, nv, widx, 0)
            gather_round(wf)

            def fin(v, _):
                word = gbuf[v // 8, pl.ds((v % 8) * 16, 16)]
                se = seb[pl.ds(v * 16, 16)]
                cb = cbb[pl.ds(v * 16, 16)]
                pos = posb[pl.ds(v * 16, 16)]
                tt = tbb[pl.ds(v * 16, 16)]
                kloc = se - cb
                for t in range(15):
                    word = jnp.where(t < kloc, word & (word - 1), word)
                bb = word & (-word)
                ee = (jnp.where((bb & 0xAAAA) != 0, 1, 0)
                      + jnp.where((bb & 0xCCCC) != 0, 2, 0)
                      + jnp.where((bb & 0xF0F0) != 0, 4, 0)
                      + jnp.where((bb & 0xFF00) != 0, 8, 0))
                fp = v * 16 + lane
                iq = fp >> log2ns
                row = q0 + iq
                fa = (row >> log2m) * n_pts
                val = pos * 16 + ee + fa
                obuf[pl.ds(v * 16, 16)] = jnp.where(tt > 0, val, fa)
                return 0

            lax.fori_loop(0, nv, fin, 0)
            pltpu.sync_copy(obuf, out.at[pl.ds(q0 * ns, lanes)])
            return 0

        lax.fori_loop(0, nbatches, batch, 0)

    return k(words_flat, c_flat)


def _mlp_pool_body(ns, cin, nlayers, h_ref, q_ref, *rest):
    w_refs = rest[:nlayers]
    b_refs = rest[nlayers:2 * nlayers]
    o_ref = rest[2 * nlayers]
    qb = h_ref.shape[0]
    hg = h_ref[...]
    q = q_ref[...]
    hx = hg[:, :, 0:3] - q[:, None, :]
    h = jnp.concatenate([hx, hg[:, :, 3:cin]], -1).reshape(qb * ns, cin)
    for i in range(nlayers):
        w = w_refs[i][...]
        b = b_refs[i][...]
        h = jnp.dot(h, w, preferred_element_type=jnp.float32) + b
        h = jnp.maximum(h, 0.0)
    c3 = h.shape[-1]
    o_ref[...] = jnp.max(h.reshape(qb, ns, c3), axis=1)


def _mlp_pool(h, q, cin, convs):
    rows, ns, dp = h.shape
    qb = 128
    nlayers = len(convs)
    wts = [jnp.transpose(c["W"]) for c in convs]
    bs = [c["b"][None, :] for c in convs]
    cout = convs[-1]["W"].shape[0]
    in_specs = [pl.BlockSpec((qb, ns, dp), lambda r: (r, 0, 0)),
                pl.BlockSpec((qb, 3), lambda r: (r, 0))]
    for w in wts:
        in_specs.append(pl.BlockSpec(w.shape, lambda r: (0, 0)))
    for b in bs:
        in_specs.append(pl.BlockSpec(b.shape, lambda r: (0, 0)))
    return pl.pallas_call(
        functools.partial(_mlp_pool_body, ns, cin, nlayers),
        grid=(rows // qb,),
        in_specs=in_specs,
        out_specs=pl.BlockSpec((qb, cout), lambda r: (r, 0)),
        out_shape=jax.ShapeDtypeStruct((rows, cout), jnp.float32),
    )(h, q, *wts, *bs)


def _matmul_relu_body(h_ref, w_ref, b_ref, o_ref):
    h = h_ref[...]
    o = jnp.dot(h, w_ref[...], preferred_element_type=jnp.float32) + b_ref[...]
    o_ref[...] = jnp.maximum(o, 0.0)


def _matmul_relu(h, w, b):
    rows, cin = h.shape
    qb = 256
    wt = jnp.transpose(w)
    cout = w.shape[0]
    return pl.pallas_call(
        _matmul_relu_body,
        grid=(rows // qb,),
        in_specs=[
            pl.BlockSpec((qb, cin), lambda r: (r, 0)),
            pl.BlockSpec(wt.shape, lambda r: (0, 0)),
            pl.BlockSpec((1, cout), lambda r: (0, 0)),
        ],
        out_specs=pl.BlockSpec((qb, cout), lambda r: (r, 0)),
        out_shape=jax.ShapeDtypeStruct((rows, cout), jnp.float32),
    )(h, wt, b[None, :])


def _sc_gather(table, idx):
    rt, d = table.shape
    r = idx.shape[0]
    mesh = plsc.VectorSubcoreMesh(core_axis_name="c", subcore_axis_name="s")
    info = plsc.get_sparse_core_info()
    nw = info.num_cores * info.num_subcores
    rpw = r // nw
    ch = min(128, rpw)
    nch = rpw // ch

    grp = 1
    for g in (4, 2, 1):
        if nch % g == 0:
            grp = g
            break

    @functools.partial(
        pl.kernel, mesh=mesh,
        out_type=jax.ShapeDtypeStruct((r, d), jnp.float32),
        scratch_types=[
            pltpu.VMEM((grp, ch), jnp.int32),
            pltpu.VMEM((grp, ch, d), jnp.float32),
            pltpu.SemaphoreType.DMA,
            pltpu.SemaphoreType.DMA,
            pltpu.SemaphoreType.DMA,
        ],
        compiler_params=pltpu.CompilerParams(use_tc_tiling_on_sc=False),
    )
    def k(tab, ih, out, idxv, rows_v, sem1, sem2, sem3):
        wid = lax.axis_index("s") * info.num_cores + lax.axis_index("c")
        base = wid * rpw

        def chunk(c, _):
            b0 = base + c * (ch * grp)
            cps = [pltpu.async_copy(ih.at[pl.ds(b0 + t * ch, ch)],
                                    idxv.at[t], sem1) for t in range(grp)]
            for cp in cps:
                cp.wait()
            cps = [pltpu.async_copy(tab.at[idxv.at[t]], rows_v.at[t], sem2)
                   for t in range(grp)]
            for cp in cps:
                cp.wait()
            cps = [pltpu.async_copy(rows_v.at[t],
                                    out.at[pl.ds(b0 + t * ch, ch)], sem3)
                   for t in range(grp)]
            for cp in cps:
                cp.wait()
            return 0

        lax.fori_loop(0, nch // grp, chunk, 0)

    return k(table, idx)


def _pad_table(x, dp):
    rt, d = x.shape
    if d == dp:
        return x
    return jnp.concatenate(
        [x, jnp.zeros((rt, dp - d), jnp.float32)], axis=-1)


def _topk_rank_body(jt, s_row_ref, s_col_ref, o_ref):
    b, ti, tj = pl.program_id(0), pl.program_id(1), pl.program_id(2)
    si = s_row_ref[0, :, :]
    sj = s_col_ref[0, :, :]
    ibase = ti * 128
    jbase = tj * jt
    iio = lax.broadcasted_iota(jnp.int32, (1, 128), 1) + ibase
    jio = lax.broadcasted_iota(jnp.int32, (jt, 1), 0) + jbase
    gt = (sj > si) | ((sj == si) & (jio < iio))
    cnt = jnp.sum(gt.astype(jnp.float32), axis=0, keepdims=True)
    prev = jnp.where(tj == 0, jnp.zeros((1, 128), jnp.float32), o_ref[0, :, :])
    o_ref[0, :, :] = prev + cnt


def _topk_sel_body(jt, m, rank_ref, o_ref):
    b, tp, tj = pl.program_id(0), pl.program_id(1), pl.program_id(2)
    rk = rank_ref[0, :, :]
    pio = lax.broadcasted_iota(jnp.int32, (1, 128), 1) + tp * 128
    ji = lax.broadcasted_iota(jnp.int32, (jt, 1), 0) + tj * jt + b * m
    hit = (rk == pio.astype(jnp.float32))
    contrib = jnp.sum(jnp.where(hit, ji.astype(jnp.float32), 0.0),
                      axis=0, keepdims=True)
    prev = jnp.where(tj == 0, jnp.zeros((1, 128), jnp.float32), o_ref[0, :, :])
    o_ref[0, :, :] = prev + contrib


def _topk_pallas(scores, k):
    b, m = scores.shape
    jt = 512
    s_row = scores[:, None, :]
    s_col = scores[:, :, None]
    rank = pl.pallas_call(
        functools.partial(_topk_rank_body, jt),
        grid=(b, m // 128, m // jt),
        in_specs=[
            pl.BlockSpec((1, 1, 128), lambda bb, i, j: (bb, 0, i)),
            pl.BlockSpec((1, jt, 1), lambda bb, i, j: (bb, j, 0)),
        ],
        out_specs=pl.BlockSpec((1, 1, 128), lambda bb, i, j: (bb, 0, i)),
        out_shape=jax.ShapeDtypeStruct((b, 1, m), jnp.float32),
    )(s_row, s_col)
    rank_col = jnp.transpose(rank, (0, 2, 1))
    sel = pl.pallas_call(
        functools.partial(_topk_sel_body, jt, m),
        grid=(b, k // 128, m // jt),
        in_specs=[
            pl.BlockSpec((1, jt, 1), lambda bb, p, j: (bb, j, 0)),
        ],
        out_specs=pl.BlockSpec((1, 1, 128), lambda bb, p, j: (bb, 0, p)),
        out_shape=jax.ShapeDtypeStruct((b, 1, k), jnp.float32),
    )(rank_col)
    return sel.reshape(b * k).astype(jnp.int32)


def _matmul_bias_body(h_ref, w_ref, b_ref, o_ref):
    h = h_ref[...]
    o_ref[...] = jnp.dot(h, w_ref[...],
                         preferred_element_type=jnp.float32) + b_ref[...]


def _matmul_bias(h, w, b):
    rows, cin = h.shape
    qb = 256
    wt = jnp.transpose(w)
    cout = w.shape[0]
    return pl.pallas_call(
        _matmul_bias_body,
        grid=(rows // qb,),
        in_specs=[
            pl.BlockSpec((qb, cin), lambda r: (r, 0)),
            pl.BlockSpec(wt.shape, lambda r: (0, 0)),
            pl.BlockSpec((1, cout), lambda r: (0, 0)),
        ],
        out_specs=pl.BlockSpec((qb, cout), lambda r: (r, 0)),
        out_shape=jax.ShapeDtypeStruct((rows, cout), jnp.float32),
    )(h, wt, b[None, :])


def _ball_query(dists, radius, nsample):
    n = dists.shape[-1]
    keyv = jnp.where(dists <= radius * radius,
                     jnp.arange(n, dtype=jnp.int32)[None, None, :], n)
    neg, _ = lax.top_k(-keyv, nsample)
    idx = -neg
    first = idx[:, :, :1]
    idx = jnp.where(idx == n, first, idx)
    idx = jnp.where(idx == n, 0, idx)
    return idx


def _identity_pallas(x):
    def body(x_ref, o_ref):
        o_ref[...] = x_ref[...]

    return pl.pallas_call(
        body,
        out_shape=jax.ShapeDtypeStruct(x.shape, x.dtype),
    )(x)


def _sa(points, feats_t, lp, li):
    npoint = _NUM_POINTS[li]
    b, n, _ = points.shape
    c = feats_t.shape[-1]
    cls_preds = None
    if _SAMPLING[li] == "ctr_aware":
        logits = _matmul_bias(feats_t.reshape(b * n, c),
                              lp["cls"]["W"], lp["cls"]["b"]).reshape(b, n, 3)
        scores = jnp.max(logits, -1)
        sel = _topk_pallas(scores, npoint)
        cls_preds = jnp.transpose(logits, (0, 2, 1))
        pts_pad = _pad_table(points.reshape(b * n, 3), 16)
        new_xyz = _sc_gather(pts_pad, sel)[:, :3].reshape(b, npoint, 3)
    else:
        new_xyz = _fps_pallas(lax.stop_gradient(points), npoint)
    pts_t3 = jnp.transpose(points, (0, 2, 1))
    words = _bq_masks(new_xyz, pts_t3, _RADII[li][0], _RADII[li][1])
    cin = 3 + c
    dp = -(-cin // 16) * 16
    tab = _pad_table(
        jnp.concatenate([points, feats_t], -1).reshape(b * n, cin), dp)
    q_flat = new_xyz.reshape(b * npoint, 3)
    outs = []
    for si, (r, ns) in enumerate(zip(_RADII[li], _NEIGHBORS[li])):
        flat = _bq_extract_sc(words[si], ns, n // 16, npoint, n)
        g = _sc_gather(tab, flat).reshape(b * npoint, ns, dp)
        pooled = _mlp_pool(g, q_flat, cin, lp["mlps"][si])
        outs.append(pooled)
    cat = jnp.concatenate(outs, -1)
    new_feat = _matmul_relu(cat, lp["agg"]["W"], lp["agg"]["b"])
    new_feat = new_feat.reshape(b, npoint, -1)
    return new_xyz, new_feat, cls_preds


def kernel(points, features, params):
    feats_t = jnp.transpose(features, (0, 2, 1))
    feats_t = _identity_pallas(feats_t)
    cls_list = []
    pts_list = []
    for li in range(3):
        ip = points
        points, feats_t, cp = _sa(points, feats_t, params["layers"][li], li)
        if cp is not None:
            cls_list.append(cp)
            pts_list.append(ip)
    return points, jnp.transpose(feats_t, (0, 2, 1)), cls_list, pts_list
